# baseline (speedup 1.0000x reference)
import functools

import jax
import jax.numpy as jnp
from jax import lax
from jax.experimental import pallas as pl
from jax.experimental.pallas import tpu as pltpu
from jax.experimental.pallas import tpu_sc as plsc

N = 10000
E = 320000
D = 128
H = 128
R = 65

NC = 2
NS = 16
NW = NC * NS
L = 16

B = 512
EW = E // NW
CW = 80
SCE = 2000
RPS = SCE // CW
SC_CHUNKS = EW // SCE
SLOTS = 6
SLOTS4 = 7

EPAD = E + R * B
NB = EPAD // B
BRW = 32
NBP = NW * BRW

NR = N * R
NRS = 40640
NRP = NS * NRS
CR = NRP // 128

HL = R * L

NHALF = 5120
NH = N // 2
NSUB = NHALF // NS


def _wid():
    return lax.axis_index("s") * NC + lax.axis_index("c")


def _mesh():
    return plsc.VectorSubcoreMesh(core_axis_name="c", subcore_axis_name="s")


def _k1_body(et_hbm, dst_hbm, hist_hbm, cntA_hbm, cntB_hbm,
             et_b, dst_b, comp_b, ones_b, hist_v, zeros_b, cnt_sh, sem):
    c = lax.axis_index("c")
    s = lax.axis_index("s")
    w = _wid()

    def _init_ones(i, _):
        ones_b[pl.ds(i * L, L)] = jnp.ones((L,), jnp.float32)
        return 0
    lax.fori_loop(0, CW // L, _init_ones, 0)

    def _zh(i, _):
        hist_v[pl.ds(i * L, L)] = jnp.zeros((L,), jnp.int32)
        return 0
    lax.fori_loop(0, R, _zh, 0)

    def _zb(i, _):
        zeros_b[pl.ds(i * L, L)] = jnp.zeros((L,), jnp.float32)
        return 0
    lax.fori_loop(0, 8128 // L, _zb, 0)

    def _zc(i, _):
        pltpu.sync_copy(zeros_b, cnt_sh.at[pl.ds(s * NRS + i * 8128, 8128)])
        return 0
    lax.fori_loop(0, NRS // 8128, _zc, 0)
    plsc.subcore_barrier()

    def _sc(sc, _):
        base = w * EW + sc * SCE
        pltpu.sync_copy(et_hbm.at[pl.ds(base, SCE)], et_b)
        pltpu.sync_copy(dst_hbm.at[pl.ds(base, SCE)], dst_b)

        def _row(j, _):
            def _vec(v, _):
                o = pl.ds(j * CW + v * L, L)
                t = et_b[o]
                d = dst_b[o]
                comp_b[j, pl.ds(v * L, L)] = d * R + t
                idx = t * L + lax.iota(jnp.int32, L)
                plsc.addupdate_scatter(hist_v, [idx], jnp.ones((L,), jnp.int32))
                return 0
            lax.fori_loop(0, CW // L, _vec, 0)
            pltpu.sync_copy(ones_b, cnt_sh.at[comp_b.at[j]], add=True)
            return 0
        lax.fori_loop(0, RPS, _row, 0)
        return 0
    lax.fori_loop(0, SC_CHUNKS, _sc, 0)

    pltpu.sync_copy(hist_v, hist_hbm.at[pl.ds(w * HL, HL)])
    plsc.subcore_barrier()

    @pl.when(c == 0)
    def _d0():
        def _dump(i, _):
            o = s * NRS + i * 8128
            pltpu.sync_copy(cnt_sh.at[pl.ds(o, 8128)], zeros_b)
            pltpu.sync_copy(zeros_b, cntA_hbm.at[pl.ds(o, 8128)])
            return 0
        lax.fori_loop(0, NRS // 8128, _dump, 0)

    @pl.when(c == 1)
    def _d1():
        def _dump(i, _):
            o = s * NRS + i * 8128
            pltpu.sync_copy(cnt_sh.at[pl.ds(o, 8128)], zeros_b)
            pltpu.sync_copy(zeros_b, cntB_hbm.at[pl.ds(o, 8128)])
            return 0
        lax.fori_loop(0, NRS // 8128, _dump, 0)


def _k1(et, dst):
    f = functools.partial(
        pl.kernel,
        out_type=(jax.ShapeDtypeStruct((NW * HL,), jnp.int32),
                  jax.ShapeDtypeStruct((NRP,), jnp.float32),
                  jax.ShapeDtypeStruct((NRP,), jnp.float32)),
        mesh=_mesh(),
        compiler_params=pltpu.CompilerParams(needs_layout_passes=False),
        scratch_types=[
            pltpu.VMEM((SCE,), jnp.int32),
            pltpu.VMEM((SCE,), jnp.int32),
            pltpu.VMEM((RPS, CW), jnp.int32),
            pltpu.VMEM((CW,), jnp.float32),
            pltpu.VMEM((HL,), jnp.int32),
            pltpu.VMEM((8128,), jnp.float32),
            pltpu.VMEM_SHARED((NRP,), jnp.float32),
            pltpu.SemaphoreType.DMA,
        ],
    )(_k1_body)
    return f(et, dst)


def _kn_body(a_ref, b_ref, o_ref):
    o_ref[...] = 1.0 / jnp.maximum(a_ref[...] + b_ref[...], 1.0)


def _kn(cntA, cntB):
    a2 = cntA.reshape(CR, 128)
    b2 = cntB.reshape(CR, 128)
    out = pl.pallas_call(
        _kn_body,
        grid=(5,),
        in_specs=[pl.BlockSpec((CR // 5, 128), lambda i: (i, 0)),
                  pl.BlockSpec((CR // 5, 128), lambda i: (i, 0))],
        out_specs=pl.BlockSpec((CR // 5, 128), lambda i: (i, 0)),
        out_shape=jax.ShapeDtypeStruct((CR, 128), jnp.float32),
    )(a2, b2)
    return out.reshape(NRP)


def _k2_body(hist_hbm, src_hbm, dst_hbm, et_hbm, ntbl_hbm, x_hbm,
             pos_hbm, norme_hbm, g_hbm, brel_hbm,
             grid_v, off2d, padend_s, brl_b,
             et_b, src_b, dst_b, pos1, norm1, comp_b, posc, rowb,
             semr0, semr1, semr2, semr3, semr4, semr5, semc):
    w = _wid()
    semr = (semr0, semr1, semr2, semr3, semr4, semr5)

    pltpu.sync_copy(hist_hbm, grid_v)

    def _ra(r, ps):
        def _wa(wp, carry):
            acc_all, acc_pri = carry
            v = grid_v[pl.ds((wp * R + r) * L, L)]
            zero = jnp.zeros((L,), jnp.int32)
            acc_all = acc_all + v
            acc_pri = acc_pri + jnp.where(wp < w, v, zero)
            return (acc_all, acc_pri)
        acc_all, acc_pri = lax.fori_loop(
            0, NW, _wa, (jnp.zeros((L,), jnp.int32), jnp.zeros((L,), jnp.int32)))
        tt = jnp.sum(acc_all)
        sb = jnp.sum(acc_pri)
        own = grid_v[pl.ds((w * R + r) * L, L)]
        excl = plsc.cumsum(own) - own
        off2d[pl.ds(r * L, L)] = (ps + sb) + excl
        padr = jnp.bitwise_and(tt + (B - 1), -B)
        pe = ps + padr
        padend_s[r] = pe
        return pe
    lax.fori_loop(0, R, _ra, jnp.int32(0))

    def _be(v, _):
        ib = (w * BRW + v * L + lax.iota(jnp.int32, L)) * B

        def _racc(r, acc):
            pe = padend_s[r]
            return acc + jnp.where(ib >= pe, 1, 0).astype(jnp.int32)
        rel = lax.fori_loop(0, R, _racc, jnp.zeros((L,), jnp.int32))
        brl_b[pl.ds(v * L, L)] = jnp.minimum(rel, R - 1)
        return 0
    lax.fori_loop(0, BRW // L, _be, 0)
    pltpu.sync_copy(brl_b, brel_hbm.at[pl.ds(w * BRW, BRW)])

    def _sc(sc, _):
        base = w * EW + sc * SCE
        pltpu.sync_copy(et_hbm.at[pl.ds(base, SCE)], et_b)
        pltpu.sync_copy(src_hbm.at[pl.ds(base, SCE)], src_b)
        pltpu.sync_copy(dst_hbm.at[pl.ds(base, SCE)], dst_b)

        def _row(j, _):
            def _vec(v, _):
                o = pl.ds(j * CW + v * L, L)
                ov = pl.ds(v * L, L)
                t = et_b[o]
                d = dst_b[o]
                comp_b[j, ov] = d * R + t
                idx = t * L + lax.iota(jnp.int32, L)
                cur = plsc.load_gather(off2d, [idx])
                plsc.store_scatter(off2d, [idx], cur + 1)
                posc[j, ov] = cur
                pos1[o] = cur
                return 0
            lax.fori_loop(0, CW // L, _vec, 0)
            return 0
        lax.fori_loop(0, RPS, _row, 0)
        pltpu.sync_copy(pos1, pos_hbm.at[pl.ds(base, SCE)])

        def _fa(j, _):
            pltpu.make_async_copy(ntbl_hbm.at[comp_b.at[j]],
                                  norm1.at[pl.ds(j * CW, CW)], semc).start()
            return 0
        lax.fori_loop(0, RPS, _fa, 0)

        def _da(j, _):
            pltpu.make_async_copy(ntbl_hbm.at[comp_b.at[j]],
                                  norm1.at[pl.ds(j * CW, CW)], semc).wait()
            return 0
        lax.fori_loop(0, RPS, _da, 0)
        pltpu.sync_copy(norm1, norme_hbm.at[pl.ds(base, SCE)])

        def _scale(j, k):
            def _rowg(g, _):
                nv = norm1[pl.ds(j * CW + g * L, L)]
                for i2 in range(L):
                    nsc = nv[i2]

                    def _vv(v, _):
                        o = pl.ds(v * L, L)
                        rowb[k, g * L + i2, o] = rowb[k, g * L + i2, o] * nsc
                        return 0
                    lax.fori_loop(0, D // L, _vv, 0)
                return 0
            lax.fori_loop(0, CW // L, _rowg, 0)

        def _fire_g(j, k):
            pltpu.make_async_copy(
                x_hbm.at[src_b.at[pl.ds(j * CW, CW)]], rowb.at[k],
                semr[k]).start()

        def _wait_g(j, k):
            pltpu.make_async_copy(
                x_hbm.at[src_b.at[pl.ds(j * CW, CW)]], rowb.at[k],
                semr[k]).wait()

        def _fire_s(j, k):
            pltpu.make_async_copy(
                rowb.at[k], g_hbm.at[posc.at[j]], semr[k]).start()

        def _wait_s(j, k):
            pltpu.make_async_copy(
                rowb.at[k], g_hbm.at[posc.at[j]], semr[k]).wait()

        for k in range(SLOTS):
            _fire_g(k, k)

        def _grp(gi, _):
            for k in range(SLOTS):
                j = gi * SLOTS + k
                _wait_g(j, k)
                _scale(j, k)
                _fire_s(j, k)
                _wait_s(j, k)
                nj = j + SLOTS

                @pl.when(nj <= RPS - 1)
                def _():
                    _fire_g(nj, k)
            return 0
        lax.fori_loop(0, (RPS - 1) // SLOTS, _grp, 0)
        jt = ((RPS - 1) // SLOTS) * SLOTS
        for k in range(RPS - jt):
            _wait_g(jt + k, k)
            _scale(jt + k, k)
            _fire_s(jt + k, k)
            _wait_s(jt + k, k)
        return 0
    lax.fori_loop(0, SC_CHUNKS, _sc, 0)


def _k2(hist_all, src, dst, et, ntbl, x):
    f = functools.partial(
        pl.kernel,
        out_type=(jax.ShapeDtypeStruct((E,), jnp.int32),
                  jax.ShapeDtypeStruct((E,), jnp.float32),
                  jax.ShapeDtypeStruct((EPAD, D), jnp.float32),
                  jax.ShapeDtypeStruct((NBP,), jnp.int32)),
        mesh=_mesh(),
        compiler_params=pltpu.CompilerParams(needs_layout_passes=False),
        scratch_types=[
            pltpu.VMEM((NW * HL,), jnp.int32),
            pltpu.VMEM((HL,), jnp.int32),
            pltpu.SMEM((R,), jnp.int32),
            pltpu.VMEM((BRW,), jnp.int32),
            pltpu.VMEM((SCE,), jnp.int32),
            pltpu.VMEM((SCE,), jnp.int32),
            pltpu.VMEM((SCE,), jnp.int32),
            pltpu.VMEM((SCE,), jnp.int32),
            pltpu.VMEM((SCE,), jnp.float32),
            pltpu.VMEM((RPS, CW), jnp.int32),
            pltpu.VMEM((RPS, CW), jnp.int32),
            pltpu.VMEM((SLOTS, CW, D), jnp.float32),
            pltpu.SemaphoreType.DMA,
            pltpu.SemaphoreType.DMA,
            pltpu.SemaphoreType.DMA,
            pltpu.SemaphoreType.DMA,
            pltpu.SemaphoreType.DMA,
            pltpu.SemaphoreType.DMA,
            pltpu.SemaphoreType.DMA,
        ],
    )(_k2_body)
    return f(hist_all, src, dst, et, ntbl, x)


def _k3_body(brel_ref, g_ref, w_ref, m_ref):
    b = brel_ref[pl.program_id(0)]
    m_ref[...] = jnp.dot(g_ref[...], w_ref[b],
                         preferred_element_type=jnp.float32)


def _k3(blockrel, G, W_rel):
    grid_spec = pltpu.PrefetchScalarGridSpec(
        num_scalar_prefetch=1,
        grid=(NB,),
        in_specs=[
            pl.BlockSpec((B, D), lambda i, brel: (i, 0)),
            pl.BlockSpec((R, D, H), lambda i, brel: (0, 0, 0)),
        ],
        out_specs=pl.BlockSpec((B, H), lambda i, brel: (i, 0)),
    )
    return pl.pallas_call(
        _k3_body,
        grid_spec=grid_spec,
        out_shape=jax.ShapeDtypeStruct((EPAD, H), jnp.float32),
    )(blockrel, G, W_rel)


def _k4_body(pos_hbm, dst_hbm, m_hbm, outp_hbm,
             pos_b, dst_b, dstc, rowb, zrow, out_sh,
             semr0, semr1, semr2, semr3, semr4, semr5, semr6, semr7):
    c = lax.axis_index("c")
    s = lax.axis_index("s")
    w = _wid()
    semr = (semr0, semr1, semr2, semr3, semr4, semr5, semr6, semr7)

    for hp in range(2):
        def _zr(i, _):
            def _zv(v, _):
                zrow[i, pl.ds(v * L, L)] = jnp.zeros((L,), jnp.float32)
                return 0
            lax.fori_loop(0, H // L, _zv, 0)
            return 0
        lax.fori_loop(0, 16, _zr, 0)

        def _zo(i, _):
            pltpu.sync_copy(zrow, out_sh.at[pl.ds(s * NSUB + i * 16, 16)])
            return 0
        lax.fori_loop(0, NSUB // 16, _zo, 0)
        plsc.subcore_barrier()

        def _sc(sc, _):
            base = w * EW + sc * SCE
            pltpu.sync_copy(pos_hbm.at[pl.ds(base, SCE)], pos_b)
            pltpu.sync_copy(dst_hbm.at[pl.ds(base, SCE)], dst_b)

            def _st(j, _):
                def _sv(v, _):
                    d = dst_b[pl.ds(j * CW + v * L, L)]
                    loc = d - hp * NH
                    valid = (loc >= 0) & (loc < NH)
                    trash = NH + jnp.bitwise_and(d, 63)
                    dstc[j, pl.ds(v * L, L)] = jnp.where(valid, loc, trash)
                    return 0
                lax.fori_loop(0, CW // L, _sv, 0)
                return 0
            lax.fori_loop(0, RPS, _st, 0)

            def _fire_g(j, k):
                pltpu.make_async_copy(
                    m_hbm.at[pos_b.at[pl.ds(j * CW, CW)]], rowb.at[k],
                    semr[k]).start()

            def _wait_g(j, k):
                pltpu.make_async_copy(
                    m_hbm.at[pos_b.at[pl.ds(j * CW, CW)]], rowb.at[k],
                    semr[k]).wait()

            def _fire_a(j, k):
                pltpu.make_async_copy(
                    rowb.at[k], out_sh.at[dstc.at[j]],
                    semr[k]).start(add=True)

            def _wait_a(j, k):
                pltpu.make_async_copy(
                    rowb.at[k], out_sh.at[dstc.at[j]],
                    semr[k]).wait()

            for k in range(SLOTS4):
                _fire_g(k, k)

            def _grp(gi, _):
                for k in range(SLOTS4):
                    j = gi * SLOTS4 + k
                    _wait_g(j, k)
                    _fire_a(j, k)
                    _wait_a(j, k)
                    nj = j + SLOTS4

                    @pl.when(nj <= RPS - 1)
                    def _():
                        _fire_g(nj, k)
                return 0
            lax.fori_loop(0, (RPS - 1) // SLOTS4, _grp, 0)
            jt = ((RPS - 1) // SLOTS4) * SLOTS4
            for k in range(RPS - jt):
                _wait_g(jt + k, k)
                _fire_a(jt + k, k)
                _wait_a(jt + k, k)
            return 0
        lax.fori_loop(0, SC_CHUNKS, _sc, 0)

        plsc.subcore_barrier()

        def _dump(i, _):
            pltpu.sync_copy(out_sh.at[pl.ds(s * NSUB + i * 16, 16)], zrow)
            pltpu.sync_copy(
                zrow,
                outp_hbm.at[pl.ds((c * 2 + hp) * NHALF + s * NSUB + i * 16, 16)])
            return 0
        lax.fori_loop(0, NSUB // 16, _dump, 0)
        plsc.subcore_barrier()


def _k4(pos, dst, M):
    f = functools.partial(
        pl.kernel,
        out_type=jax.ShapeDtypeStruct((4 * NHALF, H), jnp.float32),
        mesh=_mesh(),
        compiler_params=pltpu.CompilerParams(needs_layout_passes=False),
        scratch_types=[
            pltpu.VMEM((SCE,), jnp.int32),
            pltpu.VMEM((SCE,), jnp.int32),
            pltpu.VMEM((RPS, CW), jnp.int32),
            pltpu.VMEM((SLOTS4, CW, H), jnp.float32),
            pltpu.VMEM((16, H), jnp.float32),
            pltpu.VMEM_SHARED((NHALF, H), jnp.float32),
            pltpu.SemaphoreType.DMA,
            pltpu.SemaphoreType.DMA,
            pltpu.SemaphoreType.DMA,
            pltpu.SemaphoreType.DMA,
            pltpu.SemaphoreType.DMA,
            pltpu.SemaphoreType.DMA,
            pltpu.SemaphoreType.DMA,
            pltpu.SemaphoreType.DMA,
        ],
    )(_k4_body)
    return f(pos, dst, M)


def _k5_body(x_ref, a0_ref, a1_ref, wr_ref, wq_ref, bc_ref, br_ref, o_ref):
    xx = x_ref[...]
    out = (a0_ref[...] + a1_ref[...]
           + jnp.dot(xx, wr_ref[...], preferred_element_type=jnp.float32)
           + bc_ref[...])
    res = jnp.dot(xx, wq_ref[...], preferred_element_type=jnp.float32) + br_ref[...]
    o_ref[...] = jnp.maximum(out, 0.0) + jnp.maximum(res, 0.0)


def _k5(x, a0, a1, W_root, W_res, b_conv, b_res):
    BN = 1000
    return pl.pallas_call(
        _k5_body,
        grid=(N // BN,),
        in_specs=[
            pl.BlockSpec((BN, D), lambda i: (i, 0)),
            pl.BlockSpec((BN, H), lambda i: (i, 0)),
            pl.BlockSpec((BN, H), lambda i: (i, 0)),
            pl.BlockSpec((D, H), lambda i: (0, 0)),
            pl.BlockSpec((D, H), lambda i: (0, 0)),
            pl.BlockSpec((1, H), lambda i: (0, 0)),
            pl.BlockSpec((1, H), lambda i: (0, 0)),
        ],
        out_specs=pl.BlockSpec((BN, H), lambda i: (i, 0)),
        out_shape=jax.ShapeDtypeStruct((N, H), jnp.float32),
    )(x, a0, a1, W_root, W_res, b_conv.reshape(1, H), b_res.reshape(1, H))


def kernel(x, edge_index, edge_type, W_rel, W_root, b_conv, W_res, b_res):
    src = edge_index[0]
    dst = edge_index[1]

    hist_all, cntA, cntB = _k1(edge_type, dst)
    ntbl = _kn(cntA, cntB)
    pos, normE, G, blockrel = _k2(hist_all, src, dst, edge_type, ntbl, x)
    M = _k3(blockrel, G, W_rel)
    outp = _k4(pos, dst, M)
    a0 = jnp.concatenate([outp[0:NH], outp[NHALF:NHALF + NH]])
    a1 = jnp.concatenate([outp[2 * NHALF:2 * NHALF + NH],
                          outp[3 * NHALF:3 * NHALF + NH]])
    return _k5(x, a0, a1, W_root, W_res, b_conv, b_res)


# speedup vs baseline: 1.0039x; 1.0039x over previous
import functools

import jax
import jax.numpy as jnp
from jax import lax
from jax.experimental import pallas as pl
from jax.experimental.pallas import tpu as pltpu
from jax.experimental.pallas import tpu_sc as plsc

N = 10000
E = 320000
D = 128
H = 128
R = 65

NC = 2
NS = 16
NW = NC * NS
L = 16

B = 512
EW = E // NW
CW = 80
SCE = 2000
RPS = SCE // CW
SC_CHUNKS = EW // SCE
SLOTS = 6
SLOTS4 = 6

EPAD = E + R * B
NB = EPAD // B
BRW = 32
NBP = NW * BRW

NR = N * R
NRS = 40640
NRP = NS * NRS
CR = NRP // 128

HL = R * L

NHALF = 5120
NH = N // 2
NSUB = NHALF // NS


def _wid():
    return lax.axis_index("s") * NC + lax.axis_index("c")


def _mesh():
    return plsc.VectorSubcoreMesh(core_axis_name="c", subcore_axis_name="s")


def _k1_body(et_hbm, dst_hbm, hist_hbm, cntA_hbm, cntB_hbm,
             et_b, dst_b, comp_b, ones_b, hist_v, zeros_b, cnt_sh, sem):
    c = lax.axis_index("c")
    s = lax.axis_index("s")
    w = _wid()

    def _init_ones(i, _):
        ones_b[pl.ds(i * L, L)] = jnp.ones((L,), jnp.float32)
        return 0
    lax.fori_loop(0, CW // L, _init_ones, 0)

    def _zh(i, _):
        hist_v[pl.ds(i * L, L)] = jnp.zeros((L,), jnp.int32)
        return 0
    lax.fori_loop(0, R, _zh, 0)

    def _zb(i, _):
        zeros_b[pl.ds(i * L, L)] = jnp.zeros((L,), jnp.float32)
        return 0
    lax.fori_loop(0, 8128 // L, _zb, 0)

    def _zc(i, _):
        pltpu.sync_copy(zeros_b, cnt_sh.at[pl.ds(s * NRS + i * 8128, 8128)])
        return 0
    lax.fori_loop(0, NRS // 8128, _zc, 0)
    plsc.subcore_barrier()

    def _sc(sc, _):
        base = w * EW + sc * SCE
        pltpu.sync_copy(et_hbm.at[pl.ds(base, SCE)], et_b)
        pltpu.sync_copy(dst_hbm.at[pl.ds(base, SCE)], dst_b)

        def _row(j, _):
            def _vec(v, _):
                o = pl.ds(j * CW + v * L, L)
                t = et_b[o]
                d = dst_b[o]
                comp_b[j, pl.ds(v * L, L)] = d * R + t
                idx = t * L + lax.iota(jnp.int32, L)
                plsc.addupdate_scatter(hist_v, [idx], jnp.ones((L,), jnp.int32))
                return 0
            lax.fori_loop(0, CW // L, _vec, 0)
            pltpu.sync_copy(ones_b, cnt_sh.at[comp_b.at[j]], add=True)
            return 0
        lax.fori_loop(0, RPS, _row, 0)
        return 0
    lax.fori_loop(0, SC_CHUNKS, _sc, 0)

    pltpu.sync_copy(hist_v, hist_hbm.at[pl.ds(w * HL, HL)])
    plsc.subcore_barrier()

    @pl.when(c == 0)
    def _d0():
        def _dump(i, _):
            o = s * NRS + i * 8128
            pltpu.sync_copy(cnt_sh.at[pl.ds(o, 8128)], zeros_b)
            pltpu.sync_copy(zeros_b, cntA_hbm.at[pl.ds(o, 8128)])
            return 0
        lax.fori_loop(0, NRS // 8128, _dump, 0)

    @pl.when(c == 1)
    def _d1():
        def _dump(i, _):
            o = s * NRS + i * 8128
            pltpu.sync_copy(cnt_sh.at[pl.ds(o, 8128)], zeros_b)
            pltpu.sync_copy(zeros_b, cntB_hbm.at[pl.ds(o, 8128)])
            return 0
        lax.fori_loop(0, NRS // 8128, _dump, 0)


def _k1(et, dst):
    f = functools.partial(
        pl.kernel,
        out_type=(jax.ShapeDtypeStruct((NW * HL,), jnp.int32),
                  jax.ShapeDtypeStruct((NRP,), jnp.float32),
                  jax.ShapeDtypeStruct((NRP,), jnp.float32)),
        mesh=_mesh(),
        compiler_params=pltpu.CompilerParams(needs_layout_passes=False),
        scratch_types=[
            pltpu.VMEM((SCE,), jnp.int32),
            pltpu.VMEM((SCE,), jnp.int32),
            pltpu.VMEM((RPS, CW), jnp.int32),
            pltpu.VMEM((CW,), jnp.float32),
            pltpu.VMEM((HL,), jnp.int32),
            pltpu.VMEM((8128,), jnp.float32),
            pltpu.VMEM_SHARED((NRP,), jnp.float32),
            pltpu.SemaphoreType.DMA,
        ],
    )(_k1_body)
    return f(et, dst)


def _kn_body(a_ref, b_ref, o_ref):
    o_ref[...] = 1.0 / jnp.maximum(a_ref[...] + b_ref[...], 1.0)


def _kn(cntA, cntB):
    a2 = cntA.reshape(CR, 128)
    b2 = cntB.reshape(CR, 128)
    out = pl.pallas_call(
        _kn_body,
        grid=(5,),
        in_specs=[pl.BlockSpec((CR // 5, 128), lambda i: (i, 0)),
                  pl.BlockSpec((CR // 5, 128), lambda i: (i, 0))],
        out_specs=pl.BlockSpec((CR // 5, 128), lambda i: (i, 0)),
        out_shape=jax.ShapeDtypeStruct((CR, 128), jnp.float32),
    )(a2, b2)
    return out.reshape(NRP)


def _k2_body(hist_hbm, src_hbm, dst_hbm, et_hbm, ntbl_hbm, x_hbm,
             pos_hbm, norme_hbm, g_hbm, brel_hbm,
             grid_v, off2d, padend_s, brl_b,
             et_b, src_b, dst_b, pos1, norm1, comp_b, posc, rowb,
             semr0, semr1, semr2, semr3, semr4, semr5, semc):
    w = _wid()
    semr = (semr0, semr1, semr2, semr3, semr4, semr5)

    pltpu.sync_copy(hist_hbm, grid_v)

    def _ra(r, ps):
        def _wa(wp, carry):
            acc_all, acc_pri = carry
            v = grid_v[pl.ds((wp * R + r) * L, L)]
            zero = jnp.zeros((L,), jnp.int32)
            acc_all = acc_all + v
            acc_pri = acc_pri + jnp.where(wp < w, v, zero)
            return (acc_all, acc_pri)
        acc_all, acc_pri = lax.fori_loop(
            0, NW, _wa, (jnp.zeros((L,), jnp.int32), jnp.zeros((L,), jnp.int32)))
        tt = jnp.sum(acc_all)
        sb = jnp.sum(acc_pri)
        own = grid_v[pl.ds((w * R + r) * L, L)]
        excl = plsc.cumsum(own) - own
        off2d[pl.ds(r * L, L)] = (ps + sb) + excl
        padr = jnp.bitwise_and(tt + (B - 1), -B)
        pe = ps + padr
        padend_s[r] = pe
        return pe
    lax.fori_loop(0, R, _ra, jnp.int32(0))

    def _be(v, _):
        ib = (w * BRW + v * L + lax.iota(jnp.int32, L)) * B

        def _racc(r, acc):
            pe = padend_s[r]
            return acc + jnp.where(ib >= pe, 1, 0).astype(jnp.int32)
        rel = lax.fori_loop(0, R, _racc, jnp.zeros((L,), jnp.int32))
        brl_b[pl.ds(v * L, L)] = jnp.minimum(rel, R - 1)
        return 0
    lax.fori_loop(0, BRW // L, _be, 0)
    pltpu.sync_copy(brl_b, brel_hbm.at[pl.ds(w * BRW, BRW)])

    def _sc(sc, _):
        base = w * EW + sc * SCE
        pltpu.sync_copy(et_hbm.at[pl.ds(base, SCE)], et_b)
        pltpu.sync_copy(src_hbm.at[pl.ds(base, SCE)], src_b)
        pltpu.sync_copy(dst_hbm.at[pl.ds(base, SCE)], dst_b)

        def _row(j, _):
            def _vec(v, _):
                o = pl.ds(j * CW + v * L, L)
                ov = pl.ds(v * L, L)
                t = et_b[o]
                d = dst_b[o]
                comp_b[j, ov] = d * R + t
                idx = t * L + lax.iota(jnp.int32, L)
                cur = plsc.load_gather(off2d, [idx])
                plsc.store_scatter(off2d, [idx], cur + 1)
                posc[j, ov] = cur
                pos1[o] = cur
                return 0
            lax.fori_loop(0, CW // L, _vec, 0)
            return 0
        lax.fori_loop(0, RPS, _row, 0)
        pltpu.sync_copy(pos1, pos_hbm.at[pl.ds(base, SCE)])

        def _fa(j, _):
            pltpu.make_async_copy(ntbl_hbm.at[comp_b.at[j]],
                                  norm1.at[pl.ds(j * CW, CW)], semc).start()
            return 0
        lax.fori_loop(0, RPS, _fa, 0)

        def _da(j, _):
            pltpu.make_async_copy(ntbl_hbm.at[comp_b.at[j]],
                                  norm1.at[pl.ds(j * CW, CW)], semc).wait()
            return 0
        lax.fori_loop(0, RPS, _da, 0)
        pltpu.sync_copy(norm1, norme_hbm.at[pl.ds(base, SCE)])

        def _scale(j, k):
            def _rowg(g, _):
                nv = norm1[pl.ds(j * CW + g * L, L)]
                for i2 in range(L):
                    nsc = nv[i2]

                    def _vv(v, _):
                        o = pl.ds(v * L, L)
                        rowb[k, g * L + i2, o] = rowb[k, g * L + i2, o] * nsc
                        return 0
                    lax.fori_loop(0, D // L, _vv, 0)
                return 0
            lax.fori_loop(0, CW // L, _rowg, 0)

        def _fire_g(j, k):
            pltpu.make_async_copy(
                x_hbm.at[src_b.at[pl.ds(j * CW, CW)]], rowb.at[k],
                semr[k]).start()

        def _wait_g(j, k):
            pltpu.make_async_copy(
                x_hbm.at[src_b.at[pl.ds(j * CW, CW)]], rowb.at[k],
                semr[k]).wait()

        def _fire_s(j, k):
            pltpu.make_async_copy(
                rowb.at[k], g_hbm.at[posc.at[j]], semr[k]).start()

        def _wait_s(j, k):
            pltpu.make_async_copy(
                rowb.at[k], g_hbm.at[posc.at[j]], semr[k]).wait()

        for k in range(SLOTS):
            _fire_g(k, k)

        def _grp(gi, _):
            for k in range(SLOTS):
                j = gi * SLOTS + k
                _wait_g(j, k)
                _scale(j, k)
                _fire_s(j, k)
                _wait_s(j, k)
                nj = j + SLOTS

                @pl.when(nj <= RPS - 1)
                def _():
                    _fire_g(nj, k)
            return 0
        lax.fori_loop(0, (RPS - 1) // SLOTS, _grp, 0)
        jt = ((RPS - 1) // SLOTS) * SLOTS
        for k in range(RPS - jt):
            _wait_g(jt + k, k)
            _scale(jt + k, k)
            _fire_s(jt + k, k)
            _wait_s(jt + k, k)
        return 0
    lax.fori_loop(0, SC_CHUNKS, _sc, 0)


def _k2(hist_all, src, dst, et, ntbl, x):
    f = functools.partial(
        pl.kernel,
        out_type=(jax.ShapeDtypeStruct((E,), jnp.int32),
                  jax.ShapeDtypeStruct((E,), jnp.float32),
                  jax.ShapeDtypeStruct((EPAD, D), jnp.float32),
                  jax.ShapeDtypeStruct((NBP,), jnp.int32)),
        mesh=_mesh(),
        compiler_params=pltpu.CompilerParams(needs_layout_passes=False),
        scratch_types=[
            pltpu.VMEM((NW * HL,), jnp.int32),
            pltpu.VMEM((HL,), jnp.int32),
            pltpu.SMEM((R,), jnp.int32),
            pltpu.VMEM((BRW,), jnp.int32),
            pltpu.VMEM((SCE,), jnp.int32),
            pltpu.VMEM((SCE,), jnp.int32),
            pltpu.VMEM((SCE,), jnp.int32),
            pltpu.VMEM((SCE,), jnp.int32),
            pltpu.VMEM((SCE,), jnp.float32),
            pltpu.VMEM((RPS, CW), jnp.int32),
            pltpu.VMEM((RPS, CW), jnp.int32),
            pltpu.VMEM((SLOTS, CW, D), jnp.float32),
            pltpu.SemaphoreType.DMA,
            pltpu.SemaphoreType.DMA,
            pltpu.SemaphoreType.DMA,
            pltpu.SemaphoreType.DMA,
            pltpu.SemaphoreType.DMA,
            pltpu.SemaphoreType.DMA,
            pltpu.SemaphoreType.DMA,
        ],
    )(_k2_body)
    return f(hist_all, src, dst, et, ntbl, x)


def _k3_body(brel_ref, g_ref, w_ref, m_ref):
    b = brel_ref[pl.program_id(0)]
    m_ref[...] = jnp.dot(g_ref[...], w_ref[b],
                         preferred_element_type=jnp.float32)


def _k3(blockrel, G, W_rel):
    grid_spec = pltpu.PrefetchScalarGridSpec(
        num_scalar_prefetch=1,
        grid=(NB,),
        in_specs=[
            pl.BlockSpec((B, D), lambda i, brel: (i, 0)),
            pl.BlockSpec((R, D, H), lambda i, brel: (0, 0, 0)),
        ],
        out_specs=pl.BlockSpec((B, H), lambda i, brel: (i, 0)),
    )
    return pl.pallas_call(
        _k3_body,
        grid_spec=grid_spec,
        out_shape=jax.ShapeDtypeStruct((EPAD, H), jnp.float32),
    )(blockrel, G, W_rel)


def _k4_body(pos_hbm, dst_hbm, m_hbm, outp_hbm,
             pos_b, dst_b, dstc, rowb, zrow, out_sh,
             semr0, semr1, semr2, semr3, semr4, semr5, semr6, semr7):
    c = lax.axis_index("c")
    s = lax.axis_index("s")
    w = _wid()
    semr = (semr0, semr1, semr2, semr3, semr4, semr5, semr6, semr7)

    for hp in range(2):
        def _zr(i, _):
            def _zv(v, _):
                zrow[i, pl.ds(v * L, L)] = jnp.zeros((L,), jnp.float32)
                return 0
            lax.fori_loop(0, H // L, _zv, 0)
            return 0
        lax.fori_loop(0, 16, _zr, 0)

        def _zo(i, _):
            pltpu.sync_copy(zrow, out_sh.at[pl.ds(s * NSUB + i * 16, 16)])
            return 0
        lax.fori_loop(0, NSUB // 16, _zo, 0)
        plsc.subcore_barrier()

        def _sc(sc, _):
            base = w * EW + sc * SCE
            pltpu.sync_copy(pos_hbm.at[pl.ds(base, SCE)], pos_b)
            pltpu.sync_copy(dst_hbm.at[pl.ds(base, SCE)], dst_b)

            def _st(j, _):
                def _sv(v, _):
                    d = dst_b[pl.ds(j * CW + v * L, L)]
                    loc = d - hp * NH
                    valid = (loc >= 0) & (loc < NH)
                    trash = NH + jnp.bitwise_and(d, 63)
                    dstc[j, pl.ds(v * L, L)] = jnp.where(valid, loc, trash)
                    return 0
                lax.fori_loop(0, CW // L, _sv, 0)
                return 0
            lax.fori_loop(0, RPS, _st, 0)

            def _fire_g(j, k):
                pltpu.make_async_copy(
                    m_hbm.at[pos_b.at[pl.ds(j * CW, CW)]], rowb.at[k],
                    semr[k]).start()

            def _wait_g(j, k):
                pltpu.make_async_copy(
                    m_hbm.at[pos_b.at[pl.ds(j * CW, CW)]], rowb.at[k],
                    semr[k]).wait()

            def _fire_a(j, k):
                pltpu.make_async_copy(
                    rowb.at[k], out_sh.at[dstc.at[j]],
                    semr[k]).start(add=True)

            def _wait_a(j, k):
                pltpu.make_async_copy(
                    rowb.at[k], out_sh.at[dstc.at[j]],
                    semr[k]).wait()

            for k in range(SLOTS4):
                _fire_g(k, k)

            def _grp(gi, _):
                for k in range(SLOTS4):
                    j = gi * SLOTS4 + k
                    _wait_g(j, k)
                    _fire_a(j, k)
                    _wait_a(j, k)
                    nj = j + SLOTS4

                    @pl.when(nj <= RPS - 1)
                    def _():
                        _fire_g(nj, k)
                return 0
            lax.fori_loop(0, (RPS - 1) // SLOTS4, _grp, 0)
            jt = ((RPS - 1) // SLOTS4) * SLOTS4
            for k in range(RPS - jt):
                _wait_g(jt + k, k)
                _fire_a(jt + k, k)
                _wait_a(jt + k, k)
            return 0
        lax.fori_loop(0, SC_CHUNKS, _sc, 0)

        plsc.subcore_barrier()

        def _dump(i, _):
            pltpu.sync_copy(out_sh.at[pl.ds(s * NSUB + i * 16, 16)], zrow)
            pltpu.sync_copy(
                zrow,
                outp_hbm.at[pl.ds((c * 2 + hp) * NHALF + s * NSUB + i * 16, 16)])
            return 0
        lax.fori_loop(0, NSUB // 16, _dump, 0)
        plsc.subcore_barrier()


def _k4(pos, dst, M):
    f = functools.partial(
        pl.kernel,
        out_type=jax.ShapeDtypeStruct((4 * NHALF, H), jnp.float32),
        mesh=_mesh(),
        compiler_params=pltpu.CompilerParams(needs_layout_passes=False),
        scratch_types=[
            pltpu.VMEM((SCE,), jnp.int32),
            pltpu.VMEM((SCE,), jnp.int32),
            pltpu.VMEM((RPS, CW), jnp.int32),
            pltpu.VMEM((SLOTS4, CW, H), jnp.float32),
            pltpu.VMEM((16, H), jnp.float32),
            pltpu.VMEM_SHARED((NHALF, H), jnp.float32),
            pltpu.SemaphoreType.DMA,
            pltpu.SemaphoreType.DMA,
            pltpu.SemaphoreType.DMA,
            pltpu.SemaphoreType.DMA,
            pltpu.SemaphoreType.DMA,
            pltpu.SemaphoreType.DMA,
            pltpu.SemaphoreType.DMA,
            pltpu.SemaphoreType.DMA,
        ],
    )(_k4_body)
    return f(pos, dst, M)


def _k5_body(x_ref, a0_ref, a1_ref, wr_ref, wq_ref, bc_ref, br_ref, o_ref):
    xx = x_ref[...]
    out = (a0_ref[...] + a1_ref[...]
           + jnp.dot(xx, wr_ref[...], preferred_element_type=jnp.float32)
           + bc_ref[...])
    res = jnp.dot(xx, wq_ref[...], preferred_element_type=jnp.float32) + br_ref[...]
    o_ref[...] = jnp.maximum(out, 0.0) + jnp.maximum(res, 0.0)


def _k5(x, a0, a1, W_root, W_res, b_conv, b_res):
    BN = 1000
    return pl.pallas_call(
        _k5_body,
        grid=(N // BN,),
        in_specs=[
            pl.BlockSpec((BN, D), lambda i: (i, 0)),
            pl.BlockSpec((BN, H), lambda i: (i, 0)),
            pl.BlockSpec((BN, H), lambda i: (i, 0)),
            pl.BlockSpec((D, H), lambda i: (0, 0)),
            pl.BlockSpec((D, H), lambda i: (0, 0)),
            pl.BlockSpec((1, H), lambda i: (0, 0)),
            pl.BlockSpec((1, H), lambda i: (0, 0)),
        ],
        out_specs=pl.BlockSpec((BN, H), lambda i: (i, 0)),
        out_shape=jax.ShapeDtypeStruct((N, H), jnp.float32),
    )(x, a0, a1, W_root, W_res, b_conv.reshape(1, H), b_res.reshape(1, H))


def kernel(x, edge_index, edge_type, W_rel, W_root, b_conv, W_res, b_res):
    src = edge_index[0]
    dst = edge_index[1]

    hist_all, cntA, cntB = _k1(edge_type, dst)
    ntbl = _kn(cntA, cntB)
    pos, normE, G, blockrel = _k2(hist_all, src, dst, edge_type, ntbl, x)
    M = _k3(blockrel, G, W_rel)
    outp = _k4(pos, dst, M)
    a0 = jnp.concatenate([outp[0:NH], outp[NHALF:NHALF + NH]])
    a1 = jnp.concatenate([outp[2 * NHALF:2 * NHALF + NH],
                          outp[3 * NHALF:3 * NHALF + NH]])
    return _k5(x, a0, a1, W_root, W_res, b_conv, b_res)


# speedup vs baseline: 1.2028x; 1.1982x over previous
import functools

import jax
import jax.numpy as jnp
from jax import lax
from jax.experimental import pallas as pl
from jax.experimental.pallas import tpu as pltpu
from jax.experimental.pallas import tpu_sc as plsc

N = 10000
E = 320000
D = 128
H = 128
R = 65

NC = 2
NS = 16
NW = NC * NS
L = 16

B = 1024
EW = E // NW
CW = 80
SCE = 2000
RPS = SCE // CW
SC_CHUNKS = EW // SCE
SLOTS = 6
SLOTS4 = 6

EPAD = 387072
NB = EPAD // B
BRW = 16
NBP = NW * BRW

NR = N * R
NRS = 40640
NRP = NS * NRS
CR = NRP // 128

HL = R * L

NHALF = 5120
NH = N // 2
NSUB = NHALF // NS


def _wid():
    return lax.axis_index("s") * NC + lax.axis_index("c")


def _mesh():
    return plsc.VectorSubcoreMesh(core_axis_name="c", subcore_axis_name="s")


def _k1_body(et_hbm, dst_hbm, hist_hbm, cntA_hbm, cntB_hbm,
             et_b, dst_b, comp_b, ones_b, hist_v, zeros_b, cnt_sh, sem):
    c = lax.axis_index("c")
    s = lax.axis_index("s")
    w = _wid()

    def _init_ones(i, _):
        ones_b[pl.ds(i * L, L)] = jnp.ones((L,), jnp.float32)
        return 0
    lax.fori_loop(0, CW // L, _init_ones, 0)

    def _zh(i, _):
        hist_v[pl.ds(i * L, L)] = jnp.zeros((L,), jnp.int32)
        return 0
    lax.fori_loop(0, R, _zh, 0)

    def _zb(i, _):
        zeros_b[pl.ds(i * L, L)] = jnp.zeros((L,), jnp.float32)
        return 0
    lax.fori_loop(0, 8128 // L, _zb, 0)

    def _zc(i, _):
        pltpu.sync_copy(zeros_b, cnt_sh.at[pl.ds(s * NRS + i * 8128, 8128)])
        return 0
    lax.fori_loop(0, NRS // 8128, _zc, 0)
    plsc.subcore_barrier()

    def _sc(sc, _):
        base = w * EW + sc * SCE
        pltpu.sync_copy(et_hbm.at[pl.ds(base, SCE)], et_b)
        pltpu.sync_copy(dst_hbm.at[pl.ds(base, SCE)], dst_b)

        def _row(j, _):
            def _vec(v, _):
                o = pl.ds(j * CW + v * L, L)
                t = et_b[o]
                d = dst_b[o]
                comp_b[j, pl.ds(v * L, L)] = d * R + t
                idx = t * L + lax.iota(jnp.int32, L)
                plsc.addupdate_scatter(hist_v, [idx], jnp.ones((L,), jnp.int32))
                return 0
            lax.fori_loop(0, CW // L, _vec, 0)
            pltpu.sync_copy(ones_b, cnt_sh.at[comp_b.at[j]], add=True)
            return 0
        lax.fori_loop(0, RPS, _row, 0)
        return 0
    lax.fori_loop(0, SC_CHUNKS, _sc, 0)

    pltpu.sync_copy(hist_v, hist_hbm.at[pl.ds(w * HL, HL)])
    plsc.subcore_barrier()

    @pl.when(c == 0)
    def _d0():
        def _dump(i, _):
            o = s * NRS + i * 8128
            pltpu.sync_copy(cnt_sh.at[pl.ds(o, 8128)], zeros_b)
            pltpu.sync_copy(zeros_b, cntA_hbm.at[pl.ds(o, 8128)])
            return 0
        lax.fori_loop(0, NRS // 8128, _dump, 0)

    @pl.when(c == 1)
    def _d1():
        def _dump(i, _):
            o = s * NRS + i * 8128
            pltpu.sync_copy(cnt_sh.at[pl.ds(o, 8128)], zeros_b)
            pltpu.sync_copy(zeros_b, cntB_hbm.at[pl.ds(o, 8128)])
            return 0
        lax.fori_loop(0, NRS // 8128, _dump, 0)


def _k1(et, dst):
    f = functools.partial(
        pl.kernel,
        out_type=(jax.ShapeDtypeStruct((NW * HL,), jnp.int32),
                  jax.ShapeDtypeStruct((NRP,), jnp.float32),
                  jax.ShapeDtypeStruct((NRP,), jnp.float32)),
        mesh=_mesh(),
        compiler_params=pltpu.CompilerParams(needs_layout_passes=False),
        scratch_types=[
            pltpu.VMEM((SCE,), jnp.int32),
            pltpu.VMEM((SCE,), jnp.int32),
            pltpu.VMEM((RPS, CW), jnp.int32),
            pltpu.VMEM((CW,), jnp.float32),
            pltpu.VMEM((HL,), jnp.int32),
            pltpu.VMEM((8128,), jnp.float32),
            pltpu.VMEM_SHARED((NRP,), jnp.float32),
            pltpu.SemaphoreType.DMA,
        ],
    )(_k1_body)
    return f(et, dst)


def _kn_body(a_ref, b_ref, o_ref):
    o_ref[...] = 1.0 / jnp.maximum(a_ref[...] + b_ref[...], 1.0)


def _kn(cntA, cntB):
    a2 = cntA.reshape(CR, 128)
    b2 = cntB.reshape(CR, 128)
    out = pl.pallas_call(
        _kn_body,
        grid=(5,),
        in_specs=[pl.BlockSpec((CR // 5, 128), lambda i: (i, 0)),
                  pl.BlockSpec((CR // 5, 128), lambda i: (i, 0))],
        out_specs=pl.BlockSpec((CR // 5, 128), lambda i: (i, 0)),
        out_shape=jax.ShapeDtypeStruct((CR, 128), jnp.float32),
    )(a2, b2)
    return out.reshape(NRP)


def _k2_body(hist_hbm, src_hbm, dst_hbm, et_hbm, ntbl_hbm, x_hbm,
             pos_hbm, norme_hbm, g_hbm, brel_hbm,
             grid_v, off2d, padend_s, brl_b,
             et_b, src_b, dst_b, pos1, norm1, comp_b, posc, rowb,
             semr0, semr1, semr2, semr3, semr4, semr5, semc):
    w = _wid()
    semr = (semr0, semr1, semr2, semr3, semr4, semr5)

    pltpu.sync_copy(hist_hbm, grid_v)

    def _ra(r, ps):
        def _wa(wp, carry):
            acc_all, acc_pri = carry
            v = grid_v[pl.ds((wp * R + r) * L, L)]
            zero = jnp.zeros((L,), jnp.int32)
            acc_all = acc_all + v
            acc_pri = acc_pri + jnp.where(wp < w, v, zero)
            return (acc_all, acc_pri)
        acc_all, acc_pri = lax.fori_loop(
            0, NW, _wa, (jnp.zeros((L,), jnp.int32), jnp.zeros((L,), jnp.int32)))
        tt = jnp.sum(acc_all)
        sb = jnp.sum(acc_pri)
        own = grid_v[pl.ds((w * R + r) * L, L)]
        excl = plsc.cumsum(own) - own
        off2d[pl.ds(r * L, L)] = (ps + sb) + excl
        padr = jnp.bitwise_and(tt + (B - 1), -B)
        pe = ps + padr
        padend_s[r] = pe
        return pe
    lax.fori_loop(0, R, _ra, jnp.int32(0))

    def _be(v, _):
        ib = (w * BRW + v * L + lax.iota(jnp.int32, L)) * B

        def _racc(r, acc):
            pe = padend_s[r]
            return acc + jnp.where(ib >= pe, 1, 0).astype(jnp.int32)
        rel = lax.fori_loop(0, R, _racc, jnp.zeros((L,), jnp.int32))
        brl_b[pl.ds(v * L, L)] = jnp.minimum(rel, R - 1)
        return 0
    lax.fori_loop(0, BRW // L, _be, 0)
    pltpu.sync_copy(brl_b, brel_hbm.at[pl.ds(w * BRW, BRW)])

    def _sc(sc, _):
        base = w * EW + sc * SCE
        pltpu.sync_copy(et_hbm.at[pl.ds(base, SCE)], et_b)
        pltpu.sync_copy(src_hbm.at[pl.ds(base, SCE)], src_b)
        pltpu.sync_copy(dst_hbm.at[pl.ds(base, SCE)], dst_b)

        def _row(j, _):
            def _vec(v, _):
                o = pl.ds(j * CW + v * L, L)
                ov = pl.ds(v * L, L)
                t = et_b[o]
                d = dst_b[o]
                comp_b[j, ov] = d * R + t
                idx = t * L + lax.iota(jnp.int32, L)
                cur = plsc.load_gather(off2d, [idx])
                plsc.store_scatter(off2d, [idx], cur + 1)
                posc[j, ov] = cur
                pos1[o] = cur
                return 0
            lax.fori_loop(0, CW // L, _vec, 0)
            return 0
        lax.fori_loop(0, RPS, _row, 0)
        pltpu.sync_copy(pos1, pos_hbm.at[pl.ds(base, SCE)])

        def _fa(j, _):
            pltpu.make_async_copy(ntbl_hbm.at[comp_b.at[j]],
                                  norm1.at[pl.ds(j * CW, CW)], semc).start()
            return 0
        lax.fori_loop(0, RPS, _fa, 0)

        def _da(j, _):
            pltpu.make_async_copy(ntbl_hbm.at[comp_b.at[j]],
                                  norm1.at[pl.ds(j * CW, CW)], semc).wait()
            return 0
        lax.fori_loop(0, RPS, _da, 0)
        pltpu.sync_copy(norm1, norme_hbm.at[pl.ds(base, SCE)])

        def _scale(j, k):
            def _rowg(g, _):
                nv = norm1[pl.ds(j * CW + g * L, L)]
                for i2 in range(L):
                    nsc = nv[i2]

                    def _vv(v, _):
                        o = pl.ds(v * L, L)
                        rowb[k, g * L + i2, o] = rowb[k, g * L + i2, o] * nsc
                        return 0
                    lax.fori_loop(0, D // L, _vv, 0)
                return 0
            lax.fori_loop(0, CW // L, _rowg, 0)

        def _fire_g(j, k):
            pltpu.make_async_copy(
                x_hbm.at[src_b.at[pl.ds(j * CW, CW)]], rowb.at[k],
                semr[k]).start()

        def _wait_g(j, k):
            pltpu.make_async_copy(
                x_hbm.at[src_b.at[pl.ds(j * CW, CW)]], rowb.at[k],
                semr[k]).wait()

        def _fire_s(j, k):
            pltpu.make_async_copy(
                rowb.at[k], g_hbm.at[posc.at[j]], semr[k]).start()

        def _wait_s(j, k):
            pltpu.make_async_copy(
                rowb.at[k], g_hbm.at[posc.at[j]], semr[k]).wait()

        for k in range(SLOTS):
            _fire_g(k, k)

        def _grp(gi, _):
            for k in range(SLOTS):
                j = gi * SLOTS + k
                _wait_g(j, k)
                _scale(j, k)
                _fire_s(j, k)
                _wait_s(j, k)
                nj = j + SLOTS

                @pl.when(nj <= RPS - 1)
                def _():
                    _fire_g(nj, k)
            return 0
        lax.fori_loop(0, (RPS - 1) // SLOTS, _grp, 0)
        jt = ((RPS - 1) // SLOTS) * SLOTS
        for k in range(RPS - jt):
            _wait_g(jt + k, k)
            _scale(jt + k, k)
            _fire_s(jt + k, k)
            _wait_s(jt + k, k)
        return 0
    lax.fori_loop(0, SC_CHUNKS, _sc, 0)


def _k2(hist_all, src, dst, et, ntbl, x):
    f = functools.partial(
        pl.kernel,
        out_type=(jax.ShapeDtypeStruct((E,), jnp.int32),
                  jax.ShapeDtypeStruct((E,), jnp.float32),
                  jax.ShapeDtypeStruct((EPAD, D), jnp.float32),
                  jax.ShapeDtypeStruct((NBP,), jnp.int32)),
        mesh=_mesh(),
        compiler_params=pltpu.CompilerParams(needs_layout_passes=False),
        scratch_types=[
            pltpu.VMEM((NW * HL,), jnp.int32),
            pltpu.VMEM((HL,), jnp.int32),
            pltpu.SMEM((R,), jnp.int32),
            pltpu.VMEM((BRW,), jnp.int32),
            pltpu.VMEM((SCE,), jnp.int32),
            pltpu.VMEM((SCE,), jnp.int32),
            pltpu.VMEM((SCE,), jnp.int32),
            pltpu.VMEM((SCE,), jnp.int32),
            pltpu.VMEM((SCE,), jnp.float32),
            pltpu.VMEM((RPS, CW), jnp.int32),
            pltpu.VMEM((RPS, CW), jnp.int32),
            pltpu.VMEM((SLOTS, CW, D), jnp.float32),
            pltpu.SemaphoreType.DMA,
            pltpu.SemaphoreType.DMA,
            pltpu.SemaphoreType.DMA,
            pltpu.SemaphoreType.DMA,
            pltpu.SemaphoreType.DMA,
            pltpu.SemaphoreType.DMA,
            pltpu.SemaphoreType.DMA,
        ],
    )(_k2_body)
    return f(hist_all, src, dst, et, ntbl, x)


def _k3_body(brel_ref, g_ref, w_ref, m_ref):
    b = brel_ref[pl.program_id(0)]
    m_ref[...] = jnp.dot(g_ref[...], w_ref[b],
                         preferred_element_type=jnp.float32)


def _k3(blockrel, G, W_rel):
    grid_spec = pltpu.PrefetchScalarGridSpec(
        num_scalar_prefetch=1,
        grid=(NB,),
        in_specs=[
            pl.BlockSpec((B, D), lambda i, brel: (i, 0)),
            pl.BlockSpec((R, D, H), lambda i, brel: (0, 0, 0)),
        ],
        out_specs=pl.BlockSpec((B, H), lambda i, brel: (i, 0)),
    )
    return pl.pallas_call(
        _k3_body,
        grid_spec=grid_spec,
        out_shape=jax.ShapeDtypeStruct((EPAD, H), jnp.float32),
    )(blockrel, G, W_rel)


def _k4_body(pos_hbm, dst_hbm, m_hbm, outp_hbm,
             pos_b, dst_b, dstc, rowb, zrow, out_sh,
             semr0, semr1, semr2, semr3, semr4, semr5, semr6, semr7):
    c = lax.axis_index("c")
    s = lax.axis_index("s")
    w = _wid()
    semr = (semr0, semr1, semr2, semr3, semr4, semr5, semr6, semr7)

    for hp in range(2):
        def _zr(i, _):
            def _zv(v, _):
                zrow[i, pl.ds(v * L, L)] = jnp.zeros((L,), jnp.float32)
                return 0
            lax.fori_loop(0, H // L, _zv, 0)
            return 0
        lax.fori_loop(0, 16, _zr, 0)

        def _zo(i, _):
            pltpu.sync_copy(zrow, out_sh.at[pl.ds(s * NSUB + i * 16, 16)])
            return 0
        lax.fori_loop(0, NSUB // 16, _zo, 0)
        plsc.subcore_barrier()

        def _sc(sc, _):
            base = w * EW + sc * SCE
            pltpu.sync_copy(pos_hbm.at[pl.ds(base, SCE)], pos_b)
            pltpu.sync_copy(dst_hbm.at[pl.ds(base, SCE)], dst_b)

            def _st(j, _):
                def _sv(v, _):
                    d = dst_b[pl.ds(j * CW + v * L, L)]
                    loc = d - hp * NH
                    valid = (loc >= 0) & (loc < NH)
                    trash = NH + jnp.bitwise_and(d, 63)
                    dstc[j, pl.ds(v * L, L)] = jnp.where(valid, loc, trash)
                    return 0
                lax.fori_loop(0, CW // L, _sv, 0)
                return 0
            lax.fori_loop(0, RPS, _st, 0)

            def _fire_g(j, k):
                pltpu.make_async_copy(
                    m_hbm.at[pos_b.at[pl.ds(j * CW, CW)]], rowb.at[k],
                    semr[k]).start()

            def _wait_g(j, k):
                pltpu.make_async_copy(
                    m_hbm.at[pos_b.at[pl.ds(j * CW, CW)]], rowb.at[k],
                    semr[k]).wait()

            def _fire_a(j, k):
                pltpu.make_async_copy(
                    rowb.at[k], out_sh.at[dstc.at[j]],
                    semr[k]).start(add=True)

            def _wait_a(j, k):
                pltpu.make_async_copy(
                    rowb.at[k], out_sh.at[dstc.at[j]],
                    semr[k]).wait()

            for k in range(SLOTS4):
                _fire_g(k, k)

            def _grp(gi, _):
                for k in range(SLOTS4):
                    j = gi * SLOTS4 + k
                    _wait_g(j, k)
                    _fire_a(j, k)
                    _wait_a(j, k)
                    nj = j + SLOTS4

                    @pl.when(nj <= RPS - 1)
                    def _():
                        _fire_g(nj, k)
                return 0
            lax.fori_loop(0, (RPS - 1) // SLOTS4, _grp, 0)
            jt = ((RPS - 1) // SLOTS4) * SLOTS4
            for k in range(RPS - jt):
                _wait_g(jt + k, k)
                _fire_a(jt + k, k)
                _wait_a(jt + k, k)
            return 0
        lax.fori_loop(0, SC_CHUNKS, _sc, 0)

        plsc.subcore_barrier()

        def _dump(i, _):
            pltpu.sync_copy(out_sh.at[pl.ds(s * NSUB + i * 16, 16)], zrow)
            pltpu.sync_copy(
                zrow,
                outp_hbm.at[pl.ds((c * 2 + hp) * NHALF + s * NSUB + i * 16, 16)])
            return 0
        lax.fori_loop(0, NSUB // 16, _dump, 0)
        plsc.subcore_barrier()


def _k4(pos, dst, M):
    f = functools.partial(
        pl.kernel,
        out_type=jax.ShapeDtypeStruct((4 * NHALF, H), jnp.float32),
        mesh=_mesh(),
        compiler_params=pltpu.CompilerParams(needs_layout_passes=False),
        scratch_types=[
            pltpu.VMEM((SCE,), jnp.int32),
            pltpu.VMEM((SCE,), jnp.int32),
            pltpu.VMEM((RPS, CW), jnp.int32),
            pltpu.VMEM((SLOTS4, CW, H), jnp.float32),
            pltpu.VMEM((16, H), jnp.float32),
            pltpu.VMEM_SHARED((NHALF, H), jnp.float32),
            pltpu.SemaphoreType.DMA,
            pltpu.SemaphoreType.DMA,
            pltpu.SemaphoreType.DMA,
            pltpu.SemaphoreType.DMA,
            pltpu.SemaphoreType.DMA,
            pltpu.SemaphoreType.DMA,
            pltpu.SemaphoreType.DMA,
            pltpu.SemaphoreType.DMA,
        ],
    )(_k4_body)
    return f(pos, dst, M)


def _k5_body(x_ref, a0_ref, a1_ref, wr_ref, wq_ref, bc_ref, br_ref, o_ref):
    xx = x_ref[...]
    out = (a0_ref[...] + a1_ref[...]
           + jnp.dot(xx, wr_ref[...], preferred_element_type=jnp.float32)
           + bc_ref[...])
    res = jnp.dot(xx, wq_ref[...], preferred_element_type=jnp.float32) + br_ref[...]
    o_ref[...] = jnp.maximum(out, 0.0) + jnp.maximum(res, 0.0)


def _k5(x, a0, a1, W_root, W_res, b_conv, b_res):
    BN = 1000
    return pl.pallas_call(
        _k5_body,
        grid=(N // BN,),
        in_specs=[
            pl.BlockSpec((BN, D), lambda i: (i, 0)),
            pl.BlockSpec((BN, H), lambda i: (i, 0)),
            pl.BlockSpec((BN, H), lambda i: (i, 0)),
            pl.BlockSpec((D, H), lambda i: (0, 0)),
            pl.BlockSpec((D, H), lambda i: (0, 0)),
            pl.BlockSpec((1, H), lambda i: (0, 0)),
            pl.BlockSpec((1, H), lambda i: (0, 0)),
        ],
        out_specs=pl.BlockSpec((BN, H), lambda i: (i, 0)),
        out_shape=jax.ShapeDtypeStruct((N, H), jnp.float32),
    )(x, a0, a1, W_root, W_res, b_conv.reshape(1, H), b_res.reshape(1, H))


def kernel(x, edge_index, edge_type, W_rel, W_root, b_conv, W_res, b_res):
    src = edge_index[0]
    dst = edge_index[1]

    hist_all, cntA, cntB = _k1(edge_type, dst)
    ntbl = _kn(cntA, cntB)
    pos, normE, G, blockrel = _k2(hist_all, src, dst, edge_type, ntbl, x)
    M = _k3(blockrel, G, W_rel)
    outp = _k4(pos, dst, M)
    a0 = jnp.concatenate([outp[0:NH], outp[NHALF:NHALF + NH]])
    a1 = jnp.concatenate([outp[2 * NHALF:2 * NHALF + NH],
                          outp[3 * NHALF:3 * NHALF + NH]])
    return _k5(x, a0, a1, W_root, W_res, b_conv, b_res)


# speedup vs baseline: 1.2741x; 1.0593x over previous
import functools

import jax
import jax.numpy as jnp
from jax import lax
from jax.experimental import pallas as pl
from jax.experimental.pallas import tpu as pltpu
from jax.experimental.pallas import tpu_sc as plsc

N = 10000
E = 320000
D = 128
H = 128
R = 65

NC = 2
NS = 16
NW = NC * NS
L = 16

B = 1536
EW = E // NW
CW = 80
SCE = 2000
RPS = SCE // CW
SC_CHUNKS = EW // SCE
SLOTS = 6
SLOTS4 = 6

EPAD = 420864
NB = EPAD // B
BRW = 16
NBP = NW * BRW

NR = N * R
NRS = 40640
NRP = NS * NRS
CR = NRP // 128

HL = R * L

NHALF = 5120
NH = N // 2
NSUB = NHALF // NS


def _wid():
    return lax.axis_index("s") * NC + lax.axis_index("c")


def _mesh():
    return plsc.VectorSubcoreMesh(core_axis_name="c", subcore_axis_name="s")


def _k1_body(et_hbm, dst_hbm, hist_hbm, cntA_hbm, cntB_hbm,
             et_b, dst_b, comp_b, ones_b, hist_v, zeros_b, cnt_sh, sem):
    c = lax.axis_index("c")
    s = lax.axis_index("s")
    w = _wid()

    def _init_ones(i, _):
        ones_b[pl.ds(i * L, L)] = jnp.ones((L,), jnp.float32)
        return 0
    lax.fori_loop(0, CW // L, _init_ones, 0)

    def _zh(i, _):
        hist_v[pl.ds(i * L, L)] = jnp.zeros((L,), jnp.int32)
        return 0
    lax.fori_loop(0, R, _zh, 0)

    def _zb(i, _):
        zeros_b[pl.ds(i * L, L)] = jnp.zeros((L,), jnp.float32)
        return 0
    lax.fori_loop(0, 8128 // L, _zb, 0)

    def _zc(i, _):
        pltpu.sync_copy(zeros_b, cnt_sh.at[pl.ds(s * NRS + i * 8128, 8128)])
        return 0
    lax.fori_loop(0, NRS // 8128, _zc, 0)
    plsc.subcore_barrier()

    def _sc(sc, _):
        base = w * EW + sc * SCE
        pltpu.sync_copy(et_hbm.at[pl.ds(base, SCE)], et_b)
        pltpu.sync_copy(dst_hbm.at[pl.ds(base, SCE)], dst_b)

        def _row(j, _):
            def _vec(v, _):
                o = pl.ds(j * CW + v * L, L)
                t = et_b[o]
                d = dst_b[o]
                comp_b[j, pl.ds(v * L, L)] = d * R + t
                idx = t * L + lax.iota(jnp.int32, L)
                plsc.addupdate_scatter(hist_v, [idx], jnp.ones((L,), jnp.int32))
                return 0
            lax.fori_loop(0, CW // L, _vec, 0)
            pltpu.sync_copy(ones_b, cnt_sh.at[comp_b.at[j]], add=True)
            return 0
        lax.fori_loop(0, RPS, _row, 0)
        return 0
    lax.fori_loop(0, SC_CHUNKS, _sc, 0)

    pltpu.sync_copy(hist_v, hist_hbm.at[pl.ds(w * HL, HL)])
    plsc.subcore_barrier()

    @pl.when(c == 0)
    def _d0():
        def _dump(i, _):
            o = s * NRS + i * 8128
            pltpu.sync_copy(cnt_sh.at[pl.ds(o, 8128)], zeros_b)
            pltpu.sync_copy(zeros_b, cntA_hbm.at[pl.ds(o, 8128)])
            return 0
        lax.fori_loop(0, NRS // 8128, _dump, 0)

    @pl.when(c == 1)
    def _d1():
        def _dump(i, _):
            o = s * NRS + i * 8128
            pltpu.sync_copy(cnt_sh.at[pl.ds(o, 8128)], zeros_b)
            pltpu.sync_copy(zeros_b, cntB_hbm.at[pl.ds(o, 8128)])
            return 0
        lax.fori_loop(0, NRS // 8128, _dump, 0)


def _k1(et, dst):
    f = functools.partial(
        pl.kernel,
        out_type=(jax.ShapeDtypeStruct((NW * HL,), jnp.int32),
                  jax.ShapeDtypeStruct((NRP,), jnp.float32),
                  jax.ShapeDtypeStruct((NRP,), jnp.float32)),
        mesh=_mesh(),
        compiler_params=pltpu.CompilerParams(needs_layout_passes=False),
        scratch_types=[
            pltpu.VMEM((SCE,), jnp.int32),
            pltpu.VMEM((SCE,), jnp.int32),
            pltpu.VMEM((RPS, CW), jnp.int32),
            pltpu.VMEM((CW,), jnp.float32),
            pltpu.VMEM((HL,), jnp.int32),
            pltpu.VMEM((8128,), jnp.float32),
            pltpu.VMEM_SHARED((NRP,), jnp.float32),
            pltpu.SemaphoreType.DMA,
        ],
    )(_k1_body)
    return f(et, dst)


def _kn_body(a_ref, b_ref, o_ref):
    o_ref[...] = 1.0 / jnp.maximum(a_ref[...] + b_ref[...], 1.0)


def _kn(cntA, cntB):
    a2 = cntA.reshape(CR, 128)
    b2 = cntB.reshape(CR, 128)
    out = pl.pallas_call(
        _kn_body,
        grid=(5,),
        in_specs=[pl.BlockSpec((CR // 5, 128), lambda i: (i, 0)),
                  pl.BlockSpec((CR // 5, 128), lambda i: (i, 0))],
        out_specs=pl.BlockSpec((CR // 5, 128), lambda i: (i, 0)),
        out_shape=jax.ShapeDtypeStruct((CR, 128), jnp.float32),
    )(a2, b2)
    return out.reshape(NRP)


def _k2_body(hist_hbm, src_hbm, dst_hbm, et_hbm, ntbl_hbm, x_hbm,
             pos_hbm, norme_hbm, g_hbm, brel_hbm,
             grid_v, off2d, padend_s, brl_b,
             et_b, src_b, dst_b, pos1, norm1, comp_b, posc, rowb,
             semr0, semr1, semr2, semr3, semr4, semr5, semc):
    w = _wid()
    semr = (semr0, semr1, semr2, semr3, semr4, semr5)

    pltpu.sync_copy(hist_hbm, grid_v)

    def _ra(r, ps):
        def _wa(wp, carry):
            acc_all, acc_pri = carry
            v = grid_v[pl.ds((wp * R + r) * L, L)]
            zero = jnp.zeros((L,), jnp.int32)
            acc_all = acc_all + v
            acc_pri = acc_pri + jnp.where(wp < w, v, zero)
            return (acc_all, acc_pri)
        acc_all, acc_pri = lax.fori_loop(
            0, NW, _wa, (jnp.zeros((L,), jnp.int32), jnp.zeros((L,), jnp.int32)))
        tt = jnp.sum(acc_all)
        sb = jnp.sum(acc_pri)
        own = grid_v[pl.ds((w * R + r) * L, L)]
        excl = plsc.cumsum(own) - own
        off2d[pl.ds(r * L, L)] = (ps + sb) + excl
        padr = jnp.bitwise_and(tt + (B - 1), -B)
        pe = ps + padr
        padend_s[r] = pe
        return pe
    lax.fori_loop(0, R, _ra, jnp.int32(0))

    def _be(v, _):
        ib = (w * BRW + v * L + lax.iota(jnp.int32, L)) * B

        def _racc(r, acc):
            pe = padend_s[r]
            return acc + jnp.where(ib >= pe, 1, 0).astype(jnp.int32)
        rel = lax.fori_loop(0, R, _racc, jnp.zeros((L,), jnp.int32))
        brl_b[pl.ds(v * L, L)] = jnp.minimum(rel, R - 1)
        return 0
    lax.fori_loop(0, BRW // L, _be, 0)
    pltpu.sync_copy(brl_b, brel_hbm.at[pl.ds(w * BRW, BRW)])

    def _sc(sc, _):
        base = w * EW + sc * SCE
        pltpu.sync_copy(et_hbm.at[pl.ds(base, SCE)], et_b)
        pltpu.sync_copy(src_hbm.at[pl.ds(base, SCE)], src_b)
        pltpu.sync_copy(dst_hbm.at[pl.ds(base, SCE)], dst_b)

        def _row(j, _):
            def _vec(v, _):
                o = pl.ds(j * CW + v * L, L)
                ov = pl.ds(v * L, L)
                t = et_b[o]
                d = dst_b[o]
                comp_b[j, ov] = d * R + t
                idx = t * L + lax.iota(jnp.int32, L)
                cur = plsc.load_gather(off2d, [idx])
                plsc.store_scatter(off2d, [idx], cur + 1)
                posc[j, ov] = cur
                pos1[o] = cur
                return 0
            lax.fori_loop(0, CW // L, _vec, 0)
            return 0
        lax.fori_loop(0, RPS, _row, 0)
        pltpu.sync_copy(pos1, pos_hbm.at[pl.ds(base, SCE)])

        def _fa(j, _):
            pltpu.make_async_copy(ntbl_hbm.at[comp_b.at[j]],
                                  norm1.at[pl.ds(j * CW, CW)], semc).start()
            return 0
        lax.fori_loop(0, RPS, _fa, 0)

        def _da(j, _):
            pltpu.make_async_copy(ntbl_hbm.at[comp_b.at[j]],
                                  norm1.at[pl.ds(j * CW, CW)], semc).wait()
            return 0
        lax.fori_loop(0, RPS, _da, 0)
        pltpu.sync_copy(norm1, norme_hbm.at[pl.ds(base, SCE)])

        def _scale(j, k):
            def _rowg(g, _):
                nv = norm1[pl.ds(j * CW + g * L, L)]
                for i2 in range(L):
                    nsc = nv[i2]

                    def _vv(v, _):
                        o = pl.ds(v * L, L)
                        rowb[k, g * L + i2, o] = rowb[k, g * L + i2, o] * nsc
                        return 0
                    lax.fori_loop(0, D // L, _vv, 0)
                return 0
            lax.fori_loop(0, CW // L, _rowg, 0)

        def _fire_g(j, k):
            pltpu.make_async_copy(
                x_hbm.at[src_b.at[pl.ds(j * CW, CW)]], rowb.at[k],
                semr[k]).start()

        def _wait_g(j, k):
            pltpu.make_async_copy(
                x_hbm.at[src_b.at[pl.ds(j * CW, CW)]], rowb.at[k],
                semr[k]).wait()

        def _fire_s(j, k):
            pltpu.make_async_copy(
                rowb.at[k], g_hbm.at[posc.at[j]], semr[k]).start()

        def _wait_s(j, k):
            pltpu.make_async_copy(
                rowb.at[k], g_hbm.at[posc.at[j]], semr[k]).wait()

        for k in range(SLOTS):
            _fire_g(k, k)

        def _grp(gi, _):
            for k in range(SLOTS):
                j = gi * SLOTS + k
                _wait_g(j, k)
                _scale(j, k)
                _fire_s(j, k)
                _wait_s(j, k)
                nj = j + SLOTS

                @pl.when(nj <= RPS - 1)
                def _():
                    _fire_g(nj, k)
            return 0
        lax.fori_loop(0, (RPS - 1) // SLOTS, _grp, 0)
        jt = ((RPS - 1) // SLOTS) * SLOTS
        for k in range(RPS - jt):
            _wait_g(jt + k, k)
            _scale(jt + k, k)
            _fire_s(jt + k, k)
            _wait_s(jt + k, k)
        return 0
    lax.fori_loop(0, SC_CHUNKS, _sc, 0)


def _k2(hist_all, src, dst, et, ntbl, x):
    f = functools.partial(
        pl.kernel,
        out_type=(jax.ShapeDtypeStruct((E,), jnp.int32),
                  jax.ShapeDtypeStruct((E,), jnp.float32),
                  jax.ShapeDtypeStruct((EPAD, D), jnp.float32),
                  jax.ShapeDtypeStruct((NBP,), jnp.int32)),
        mesh=_mesh(),
        compiler_params=pltpu.CompilerParams(needs_layout_passes=False),
        scratch_types=[
            pltpu.VMEM((NW * HL,), jnp.int32),
            pltpu.VMEM((HL,), jnp.int32),
            pltpu.SMEM((R,), jnp.int32),
            pltpu.VMEM((BRW,), jnp.int32),
            pltpu.VMEM((SCE,), jnp.int32),
            pltpu.VMEM((SCE,), jnp.int32),
            pltpu.VMEM((SCE,), jnp.int32),
            pltpu.VMEM((SCE,), jnp.int32),
            pltpu.VMEM((SCE,), jnp.float32),
            pltpu.VMEM((RPS, CW), jnp.int32),
            pltpu.VMEM((RPS, CW), jnp.int32),
            pltpu.VMEM((SLOTS, CW, D), jnp.float32),
            pltpu.SemaphoreType.DMA,
            pltpu.SemaphoreType.DMA,
            pltpu.SemaphoreType.DMA,
            pltpu.SemaphoreType.DMA,
            pltpu.SemaphoreType.DMA,
            pltpu.SemaphoreType.DMA,
            pltpu.SemaphoreType.DMA,
        ],
    )(_k2_body)
    return f(hist_all, src, dst, et, ntbl, x)


def _k3_body(brel_ref, g_ref, w_ref, m_ref):
    b = brel_ref[pl.program_id(0)]
    m_ref[...] = jnp.dot(g_ref[...], w_ref[b],
                         preferred_element_type=jnp.float32)


def _k3(blockrel, G, W_rel):
    grid_spec = pltpu.PrefetchScalarGridSpec(
        num_scalar_prefetch=1,
        grid=(NB,),
        in_specs=[
            pl.BlockSpec((B, D), lambda i, brel: (i, 0)),
            pl.BlockSpec((R, D, H), lambda i, brel: (0, 0, 0)),
        ],
        out_specs=pl.BlockSpec((B, H), lambda i, brel: (i, 0)),
    )
    return pl.pallas_call(
        _k3_body,
        grid_spec=grid_spec,
        out_shape=jax.ShapeDtypeStruct((EPAD, H), jnp.float32),
    )(blockrel, G, W_rel)


def _k4_body(pos_hbm, dst_hbm, m_hbm, outp_hbm,
             pos_b, dst_b, dstc, rowb, zrow, out_sh,
             semr0, semr1, semr2, semr3, semr4, semr5, semr6, semr7):
    c = lax.axis_index("c")
    s = lax.axis_index("s")
    w = _wid()
    semr = (semr0, semr1, semr2, semr3, semr4, semr5, semr6, semr7)

    for hp in range(2):
        def _zr(i, _):
            def _zv(v, _):
                zrow[i, pl.ds(v * L, L)] = jnp.zeros((L,), jnp.float32)
                return 0
            lax.fori_loop(0, H // L, _zv, 0)
            return 0
        lax.fori_loop(0, 16, _zr, 0)

        def _zo(i, _):
            pltpu.sync_copy(zrow, out_sh.at[pl.ds(s * NSUB + i * 16, 16)])
            return 0
        lax.fori_loop(0, NSUB // 16, _zo, 0)
        plsc.subcore_barrier()

        def _sc(sc, _):
            base = w * EW + sc * SCE
            pltpu.sync_copy(pos_hbm.at[pl.ds(base, SCE)], pos_b)
            pltpu.sync_copy(dst_hbm.at[pl.ds(base, SCE)], dst_b)

            def _st(j, _):
                def _sv(v, _):
                    d = dst_b[pl.ds(j * CW + v * L, L)]
                    loc = d - hp * NH
                    valid = (loc >= 0) & (loc < NH)
                    trash = NH + jnp.bitwise_and(d, 63)
                    dstc[j, pl.ds(v * L, L)] = jnp.where(valid, loc, trash)
                    return 0
                lax.fori_loop(0, CW // L, _sv, 0)
                return 0
            lax.fori_loop(0, RPS, _st, 0)

            def _fire_g(j, k):
                pltpu.make_async_copy(
                    m_hbm.at[pos_b.at[pl.ds(j * CW, CW)]], rowb.at[k],
                    semr[k]).start()

            def _wait_g(j, k):
                pltpu.make_async_copy(
                    m_hbm.at[pos_b.at[pl.ds(j * CW, CW)]], rowb.at[k],
                    semr[k]).wait()

            def _fire_a(j, k):
                pltpu.make_async_copy(
                    rowb.at[k], out_sh.at[dstc.at[j]],
                    semr[k]).start(add=True)

            def _wait_a(j, k):
                pltpu.make_async_copy(
                    rowb.at[k], out_sh.at[dstc.at[j]],
                    semr[k]).wait()

            for k in range(SLOTS4):
                _fire_g(k, k)

            def _grp(gi, _):
                for k in range(SLOTS4):
                    j = gi * SLOTS4 + k
                    _wait_g(j, k)
                    _fire_a(j, k)
                    _wait_a(j, k)
                    nj = j + SLOTS4

                    @pl.when(nj <= RPS - 1)
                    def _():
                        _fire_g(nj, k)
                return 0
            lax.fori_loop(0, (RPS - 1) // SLOTS4, _grp, 0)
            jt = ((RPS - 1) // SLOTS4) * SLOTS4
            for k in range(RPS - jt):
                _wait_g(jt + k, k)
                _fire_a(jt + k, k)
                _wait_a(jt + k, k)
            return 0
        lax.fori_loop(0, SC_CHUNKS, _sc, 0)

        plsc.subcore_barrier()

        def _dump(i, _):
            pltpu.sync_copy(out_sh.at[pl.ds(s * NSUB + i * 16, 16)], zrow)
            pltpu.sync_copy(
                zrow,
                outp_hbm.at[pl.ds((c * 2 + hp) * NHALF + s * NSUB + i * 16, 16)])
            return 0
        lax.fori_loop(0, NSUB // 16, _dump, 0)
        plsc.subcore_barrier()


def _k4(pos, dst, M):
    f = functools.partial(
        pl.kernel,
        out_type=jax.ShapeDtypeStruct((4 * NHALF, H), jnp.float32),
        mesh=_mesh(),
        compiler_params=pltpu.CompilerParams(needs_layout_passes=False),
        scratch_types=[
            pltpu.VMEM((SCE,), jnp.int32),
            pltpu.VMEM((SCE,), jnp.int32),
            pltpu.VMEM((RPS, CW), jnp.int32),
            pltpu.VMEM((SLOTS4, CW, H), jnp.float32),
            pltpu.VMEM((16, H), jnp.float32),
            pltpu.VMEM_SHARED((NHALF, H), jnp.float32),
            pltpu.SemaphoreType.DMA,
            pltpu.SemaphoreType.DMA,
            pltpu.SemaphoreType.DMA,
            pltpu.SemaphoreType.DMA,
            pltpu.SemaphoreType.DMA,
            pltpu.SemaphoreType.DMA,
            pltpu.SemaphoreType.DMA,
            pltpu.SemaphoreType.DMA,
        ],
    )(_k4_body)
    return f(pos, dst, M)


def _k5_body(x_ref, a0_ref, a1_ref, wr_ref, wq_ref, bc_ref, br_ref, o_ref):
    xx = x_ref[...]
    out = (a0_ref[...] + a1_ref[...]
           + jnp.dot(xx, wr_ref[...], preferred_element_type=jnp.float32)
           + bc_ref[...])
    res = jnp.dot(xx, wq_ref[...], preferred_element_type=jnp.float32) + br_ref[...]
    o_ref[...] = jnp.maximum(out, 0.0) + jnp.maximum(res, 0.0)


def _k5(x, a0, a1, W_root, W_res, b_conv, b_res):
    BN = 1000
    return pl.pallas_call(
        _k5_body,
        grid=(N // BN,),
        in_specs=[
            pl.BlockSpec((BN, D), lambda i: (i, 0)),
            pl.BlockSpec((BN, H), lambda i: (i, 0)),
            pl.BlockSpec((BN, H), lambda i: (i, 0)),
            pl.BlockSpec((D, H), lambda i: (0, 0)),
            pl.BlockSpec((D, H), lambda i: (0, 0)),
            pl.BlockSpec((1, H), lambda i: (0, 0)),
            pl.BlockSpec((1, H), lambda i: (0, 0)),
        ],
        out_specs=pl.BlockSpec((BN, H), lambda i: (i, 0)),
        out_shape=jax.ShapeDtypeStruct((N, H), jnp.float32),
    )(x, a0, a1, W_root, W_res, b_conv.reshape(1, H), b_res.reshape(1, H))


def kernel(x, edge_index, edge_type, W_rel, W_root, b_conv, W_res, b_res):
    src = edge_index[0]
    dst = edge_index[1]

    hist_all, cntA, cntB = _k1(edge_type, dst)
    ntbl = _kn(cntA, cntB)
    pos, normE, G, blockrel = _k2(hist_all, src, dst, edge_type, ntbl, x)
    M = _k3(blockrel, G, W_rel)
    outp = _k4(pos, dst, M)
    a0 = jnp.concatenate([outp[0:NH], outp[NHALF:NHALF + NH]])
    a1 = jnp.concatenate([outp[2 * NHALF:2 * NHALF + NH],
                          outp[3 * NHALF:3 * NHALF + NH]])
    return _k5(x, a0, a1, W_root, W_res, b_conv, b_res)


# speedup vs baseline: 1.2952x; 1.0166x over previous
import functools

import jax
import jax.numpy as jnp
from jax import lax
from jax.experimental import pallas as pl
from jax.experimental.pallas import tpu as pltpu
from jax.experimental.pallas import tpu_sc as plsc

N = 10000
E = 320000
D = 128
H = 128
R = 65

NC = 2
NS = 16
NW = NC * NS
L = 16

B = 2048
EW = E // NW
CW = 80
SCE = 2000
RPS = SCE // CW
SC_CHUNKS = EW // SCE
SLOTS = 6
SLOTS4 = 6

EPAD = 454656
NB = EPAD // B
BRW = 16
NBP = NW * BRW

NR = N * R
NRS = 40640
NRP = NS * NRS
CR = NRP // 128

HL = R * L

NHALF = 5120
NH = N // 2
NSUB = NHALF // NS


def _wid():
    return lax.axis_index("s") * NC + lax.axis_index("c")


def _mesh():
    return plsc.VectorSubcoreMesh(core_axis_name="c", subcore_axis_name="s")


def _k1_body(et_hbm, dst_hbm, hist_hbm, cntA_hbm, cntB_hbm,
             et_b, dst_b, comp_b, ones_b, hist_v, zeros_b, cnt_sh, sem):
    c = lax.axis_index("c")
    s = lax.axis_index("s")
    w = _wid()

    def _init_ones(i, _):
        ones_b[pl.ds(i * L, L)] = jnp.ones((L,), jnp.float32)
        return 0
    lax.fori_loop(0, CW // L, _init_ones, 0)

    def _zh(i, _):
        hist_v[pl.ds(i * L, L)] = jnp.zeros((L,), jnp.int32)
        return 0
    lax.fori_loop(0, R, _zh, 0)

    def _zb(i, _):
        zeros_b[pl.ds(i * L, L)] = jnp.zeros((L,), jnp.float32)
        return 0
    lax.fori_loop(0, 8128 // L, _zb, 0)

    def _zc(i, _):
        pltpu.sync_copy(zeros_b, cnt_sh.at[pl.ds(s * NRS + i * 8128, 8128)])
        return 0
    lax.fori_loop(0, NRS // 8128, _zc, 0)
    plsc.subcore_barrier()

    def _sc(sc, _):
        base = w * EW + sc * SCE
        pltpu.sync_copy(et_hbm.at[pl.ds(base, SCE)], et_b)
        pltpu.sync_copy(dst_hbm.at[pl.ds(base, SCE)], dst_b)

        def _row(j, _):
            def _vec(v, _):
                o = pl.ds(j * CW + v * L, L)
                t = et_b[o]
                d = dst_b[o]
                comp_b[j, pl.ds(v * L, L)] = d * R + t
                idx = t * L + lax.iota(jnp.int32, L)
                plsc.addupdate_scatter(hist_v, [idx], jnp.ones((L,), jnp.int32))
                return 0
            lax.fori_loop(0, CW // L, _vec, 0)
            pltpu.sync_copy(ones_b, cnt_sh.at[comp_b.at[j]], add=True)
            return 0
        lax.fori_loop(0, RPS, _row, 0)
        return 0
    lax.fori_loop(0, SC_CHUNKS, _sc, 0)

    pltpu.sync_copy(hist_v, hist_hbm.at[pl.ds(w * HL, HL)])
    plsc.subcore_barrier()

    @pl.when(c == 0)
    def _d0():
        def _dump(i, _):
            o = s * NRS + i * 8128
            pltpu.sync_copy(cnt_sh.at[pl.ds(o, 8128)], zeros_b)
            pltpu.sync_copy(zeros_b, cntA_hbm.at[pl.ds(o, 8128)])
            return 0
        lax.fori_loop(0, NRS // 8128, _dump, 0)

    @pl.when(c == 1)
    def _d1():
        def _dump(i, _):
            o = s * NRS + i * 8128
            pltpu.sync_copy(cnt_sh.at[pl.ds(o, 8128)], zeros_b)
            pltpu.sync_copy(zeros_b, cntB_hbm.at[pl.ds(o, 8128)])
            return 0
        lax.fori_loop(0, NRS // 8128, _dump, 0)


def _k1(et, dst):
    f = functools.partial(
        pl.kernel,
        out_type=(jax.ShapeDtypeStruct((NW * HL,), jnp.int32),
                  jax.ShapeDtypeStruct((NRP,), jnp.float32),
                  jax.ShapeDtypeStruct((NRP,), jnp.float32)),
        mesh=_mesh(),
        compiler_params=pltpu.CompilerParams(needs_layout_passes=False),
        scratch_types=[
            pltpu.VMEM((SCE,), jnp.int32),
            pltpu.VMEM((SCE,), jnp.int32),
            pltpu.VMEM((RPS, CW), jnp.int32),
            pltpu.VMEM((CW,), jnp.float32),
            pltpu.VMEM((HL,), jnp.int32),
            pltpu.VMEM((8128,), jnp.float32),
            pltpu.VMEM_SHARED((NRP,), jnp.float32),
            pltpu.SemaphoreType.DMA,
        ],
    )(_k1_body)
    return f(et, dst)


def _kn_body(a_ref, b_ref, o_ref):
    o_ref[...] = 1.0 / jnp.maximum(a_ref[...] + b_ref[...], 1.0)


def _kn(cntA, cntB):
    a2 = cntA.reshape(CR, 128)
    b2 = cntB.reshape(CR, 128)
    out = pl.pallas_call(
        _kn_body,
        grid=(5,),
        in_specs=[pl.BlockSpec((CR // 5, 128), lambda i: (i, 0)),
                  pl.BlockSpec((CR // 5, 128), lambda i: (i, 0))],
        out_specs=pl.BlockSpec((CR // 5, 128), lambda i: (i, 0)),
        out_shape=jax.ShapeDtypeStruct((CR, 128), jnp.float32),
    )(a2, b2)
    return out.reshape(NRP)


def _k2_body(hist_hbm, src_hbm, dst_hbm, et_hbm, ntbl_hbm, x_hbm,
             pos_hbm, norme_hbm, g_hbm, brel_hbm,
             grid_v, off2d, padend_s, brl_b,
             et_b, src_b, dst_b, pos1, norm1, comp_b, posc, rowb,
             semr0, semr1, semr2, semr3, semr4, semr5, semc):
    w = _wid()
    semr = (semr0, semr1, semr2, semr3, semr4, semr5)

    pltpu.sync_copy(hist_hbm, grid_v)

    def _ra(r, ps):
        def _wa(wp, carry):
            acc_all, acc_pri = carry
            v = grid_v[pl.ds((wp * R + r) * L, L)]
            zero = jnp.zeros((L,), jnp.int32)
            acc_all = acc_all + v
            acc_pri = acc_pri + jnp.where(wp < w, v, zero)
            return (acc_all, acc_pri)
        acc_all, acc_pri = lax.fori_loop(
            0, NW, _wa, (jnp.zeros((L,), jnp.int32), jnp.zeros((L,), jnp.int32)))
        tt = jnp.sum(acc_all)
        sb = jnp.sum(acc_pri)
        own = grid_v[pl.ds((w * R + r) * L, L)]
        excl = plsc.cumsum(own) - own
        off2d[pl.ds(r * L, L)] = (ps + sb) + excl
        padr = jnp.bitwise_and(tt + (B - 1), -B)
        pe = ps + padr
        padend_s[r] = pe
        return pe
    lax.fori_loop(0, R, _ra, jnp.int32(0))

    def _be(v, _):
        ib = (w * BRW + v * L + lax.iota(jnp.int32, L)) * B

        def _racc(r, acc):
            pe = padend_s[r]
            return acc + jnp.where(ib >= pe, 1, 0).astype(jnp.int32)
        rel = lax.fori_loop(0, R, _racc, jnp.zeros((L,), jnp.int32))
        brl_b[pl.ds(v * L, L)] = jnp.minimum(rel, R - 1)
        return 0
    lax.fori_loop(0, BRW // L, _be, 0)
    pltpu.sync_copy(brl_b, brel_hbm.at[pl.ds(w * BRW, BRW)])

    def _sc(sc, _):
        base = w * EW + sc * SCE
        pltpu.sync_copy(et_hbm.at[pl.ds(base, SCE)], et_b)
        pltpu.sync_copy(src_hbm.at[pl.ds(base, SCE)], src_b)
        pltpu.sync_copy(dst_hbm.at[pl.ds(base, SCE)], dst_b)

        def _row(j, _):
            def _vec(v, _):
                o = pl.ds(j * CW + v * L, L)
                ov = pl.ds(v * L, L)
                t = et_b[o]
                d = dst_b[o]
                comp_b[j, ov] = d * R + t
                idx = t * L + lax.iota(jnp.int32, L)
                cur = plsc.load_gather(off2d, [idx])
                plsc.store_scatter(off2d, [idx], cur + 1)
                posc[j, ov] = cur
                pos1[o] = cur
                return 0
            lax.fori_loop(0, CW // L, _vec, 0)
            return 0
        lax.fori_loop(0, RPS, _row, 0)
        pltpu.sync_copy(pos1, pos_hbm.at[pl.ds(base, SCE)])

        def _fa(j, _):
            pltpu.make_async_copy(ntbl_hbm.at[comp_b.at[j]],
                                  norm1.at[pl.ds(j * CW, CW)], semc).start()
            return 0
        lax.fori_loop(0, RPS, _fa, 0)

        def _da(j, _):
            pltpu.make_async_copy(ntbl_hbm.at[comp_b.at[j]],
                                  norm1.at[pl.ds(j * CW, CW)], semc).wait()
            return 0
        lax.fori_loop(0, RPS, _da, 0)
        pltpu.sync_copy(norm1, norme_hbm.at[pl.ds(base, SCE)])

        def _scale(j, k):
            def _rowg(g, _):
                nv = norm1[pl.ds(j * CW + g * L, L)]
                for i2 in range(L):
                    nsc = nv[i2]

                    def _vv(v, _):
                        o = pl.ds(v * L, L)
                        rowb[k, g * L + i2, o] = rowb[k, g * L + i2, o] * nsc
                        return 0
                    lax.fori_loop(0, D // L, _vv, 0)
                return 0
            lax.fori_loop(0, CW // L, _rowg, 0)

        def _fire_g(j, k):
            pltpu.make_async_copy(
                x_hbm.at[src_b.at[pl.ds(j * CW, CW)]], rowb.at[k],
                semr[k]).start()

        def _wait_g(j, k):
            pltpu.make_async_copy(
                x_hbm.at[src_b.at[pl.ds(j * CW, CW)]], rowb.at[k],
                semr[k]).wait()

        def _fire_s(j, k):
            pltpu.make_async_copy(
                rowb.at[k], g_hbm.at[posc.at[j]], semr[k]).start()

        def _wait_s(j, k):
            pltpu.make_async_copy(
                rowb.at[k], g_hbm.at[posc.at[j]], semr[k]).wait()

        for k in range(SLOTS):
            _fire_g(k, k)

        def _grp(gi, _):
            for k in range(SLOTS):
                j = gi * SLOTS + k
                _wait_g(j, k)
                _scale(j, k)
                _fire_s(j, k)
                _wait_s(j, k)
                nj = j + SLOTS

                @pl.when(nj <= RPS - 1)
                def _():
                    _fire_g(nj, k)
            return 0
        lax.fori_loop(0, (RPS - 1) // SLOTS, _grp, 0)
        jt = ((RPS - 1) // SLOTS) * SLOTS
        for k in range(RPS - jt):
            _wait_g(jt + k, k)
            _scale(jt + k, k)
            _fire_s(jt + k, k)
            _wait_s(jt + k, k)
        return 0
    lax.fori_loop(0, SC_CHUNKS, _sc, 0)


def _k2(hist_all, src, dst, et, ntbl, x):
    f = functools.partial(
        pl.kernel,
        out_type=(jax.ShapeDtypeStruct((E,), jnp.int32),
                  jax.ShapeDtypeStruct((E,), jnp.float32),
                  jax.ShapeDtypeStruct((EPAD, D), jnp.float32),
                  jax.ShapeDtypeStruct((NBP,), jnp.int32)),
        mesh=_mesh(),
        compiler_params=pltpu.CompilerParams(needs_layout_passes=False),
        scratch_types=[
            pltpu.VMEM((NW * HL,), jnp.int32),
            pltpu.VMEM((HL,), jnp.int32),
            pltpu.SMEM((R,), jnp.int32),
            pltpu.VMEM((BRW,), jnp.int32),
            pltpu.VMEM((SCE,), jnp.int32),
            pltpu.VMEM((SCE,), jnp.int32),
            pltpu.VMEM((SCE,), jnp.int32),
            pltpu.VMEM((SCE,), jnp.int32),
            pltpu.VMEM((SCE,), jnp.float32),
            pltpu.VMEM((RPS, CW), jnp.int32),
            pltpu.VMEM((RPS, CW), jnp.int32),
            pltpu.VMEM((SLOTS, CW, D), jnp.float32),
            pltpu.SemaphoreType.DMA,
            pltpu.SemaphoreType.DMA,
            pltpu.SemaphoreType.DMA,
            pltpu.SemaphoreType.DMA,
            pltpu.SemaphoreType.DMA,
            pltpu.SemaphoreType.DMA,
            pltpu.SemaphoreType.DMA,
        ],
    )(_k2_body)
    return f(hist_all, src, dst, et, ntbl, x)


def _k3_body(brel_ref, g_ref, w_ref, m_ref):
    b = brel_ref[pl.program_id(0)]
    m_ref[...] = jnp.dot(g_ref[...], w_ref[b],
                         preferred_element_type=jnp.float32)


def _k3(blockrel, G, W_rel):
    grid_spec = pltpu.PrefetchScalarGridSpec(
        num_scalar_prefetch=1,
        grid=(NB,),
        in_specs=[
            pl.BlockSpec((B, D), lambda i, brel: (i, 0)),
            pl.BlockSpec((R, D, H), lambda i, brel: (0, 0, 0)),
        ],
        out_specs=pl.BlockSpec((B, H), lambda i, brel: (i, 0)),
    )
    return pl.pallas_call(
        _k3_body,
        grid_spec=grid_spec,
        out_shape=jax.ShapeDtypeStruct((EPAD, H), jnp.float32),
    )(blockrel, G, W_rel)


def _k4_body(pos_hbm, dst_hbm, m_hbm, outp_hbm,
             pos_b, dst_b, dstc, rowb, zrow, out_sh,
             semr0, semr1, semr2, semr3, semr4, semr5, semr6, semr7):
    c = lax.axis_index("c")
    s = lax.axis_index("s")
    w = _wid()
    semr = (semr0, semr1, semr2, semr3, semr4, semr5, semr6, semr7)

    for hp in range(2):
        def _zr(i, _):
            def _zv(v, _):
                zrow[i, pl.ds(v * L, L)] = jnp.zeros((L,), jnp.float32)
                return 0
            lax.fori_loop(0, H // L, _zv, 0)
            return 0
        lax.fori_loop(0, 16, _zr, 0)

        def _zo(i, _):
            pltpu.sync_copy(zrow, out_sh.at[pl.ds(s * NSUB + i * 16, 16)])
            return 0
        lax.fori_loop(0, NSUB // 16, _zo, 0)
        plsc.subcore_barrier()

        def _sc(sc, _):
            base = w * EW + sc * SCE
            pltpu.sync_copy(pos_hbm.at[pl.ds(base, SCE)], pos_b)
            pltpu.sync_copy(dst_hbm.at[pl.ds(base, SCE)], dst_b)

            def _st(j, _):
                def _sv(v, _):
                    d = dst_b[pl.ds(j * CW + v * L, L)]
                    loc = d - hp * NH
                    valid = (loc >= 0) & (loc < NH)
                    trash = NH + jnp.bitwise_and(d, 63)
                    dstc[j, pl.ds(v * L, L)] = jnp.where(valid, loc, trash)
                    return 0
                lax.fori_loop(0, CW // L, _sv, 0)
                return 0
            lax.fori_loop(0, RPS, _st, 0)

            def _fire_g(j, k):
                pltpu.make_async_copy(
                    m_hbm.at[pos_b.at[pl.ds(j * CW, CW)]], rowb.at[k],
                    semr[k]).start()

            def _wait_g(j, k):
                pltpu.make_async_copy(
                    m_hbm.at[pos_b.at[pl.ds(j * CW, CW)]], rowb.at[k],
                    semr[k]).wait()

            def _fire_a(j, k):
                pltpu.make_async_copy(
                    rowb.at[k], out_sh.at[dstc.at[j]],
                    semr[k]).start(add=True)

            def _wait_a(j, k):
                pltpu.make_async_copy(
                    rowb.at[k], out_sh.at[dstc.at[j]],
                    semr[k]).wait()

            for k in range(SLOTS4):
                _fire_g(k, k)

            def _grp(gi, _):
                for k in range(SLOTS4):
                    j = gi * SLOTS4 + k
                    _wait_g(j, k)
                    _fire_a(j, k)
                    _wait_a(j, k)
                    nj = j + SLOTS4

                    @pl.when(nj <= RPS - 1)
                    def _():
                        _fire_g(nj, k)
                return 0
            lax.fori_loop(0, (RPS - 1) // SLOTS4, _grp, 0)
            jt = ((RPS - 1) // SLOTS4) * SLOTS4
            for k in range(RPS - jt):
                _wait_g(jt + k, k)
                _fire_a(jt + k, k)
                _wait_a(jt + k, k)
            return 0
        lax.fori_loop(0, SC_CHUNKS, _sc, 0)

        plsc.subcore_barrier()

        def _dump(i, _):
            pltpu.sync_copy(out_sh.at[pl.ds(s * NSUB + i * 16, 16)], zrow)
            pltpu.sync_copy(
                zrow,
                outp_hbm.at[pl.ds((c * 2 + hp) * NHALF + s * NSUB + i * 16, 16)])
            return 0
        lax.fori_loop(0, NSUB // 16, _dump, 0)
        plsc.subcore_barrier()


def _k4(pos, dst, M):
    f = functools.partial(
        pl.kernel,
        out_type=jax.ShapeDtypeStruct((4 * NHALF, H), jnp.float32),
        mesh=_mesh(),
        compiler_params=pltpu.CompilerParams(needs_layout_passes=False),
        scratch_types=[
            pltpu.VMEM((SCE,), jnp.int32),
            pltpu.VMEM((SCE,), jnp.int32),
            pltpu.VMEM((RPS, CW), jnp.int32),
            pltpu.VMEM((SLOTS4, CW, H), jnp.float32),
            pltpu.VMEM((16, H), jnp.float32),
            pltpu.VMEM_SHARED((NHALF, H), jnp.float32),
            pltpu.SemaphoreType.DMA,
            pltpu.SemaphoreType.DMA,
            pltpu.SemaphoreType.DMA,
            pltpu.SemaphoreType.DMA,
            pltpu.SemaphoreType.DMA,
            pltpu.SemaphoreType.DMA,
            pltpu.SemaphoreType.DMA,
            pltpu.SemaphoreType.DMA,
        ],
    )(_k4_body)
    return f(pos, dst, M)


def _k5_body(x_ref, a0_ref, a1_ref, wr_ref, wq_ref, bc_ref, br_ref, o_ref):
    xx = x_ref[...]
    out = (a0_ref[...] + a1_ref[...]
           + jnp.dot(xx, wr_ref[...], preferred_element_type=jnp.float32)
           + bc_ref[...])
    res = jnp.dot(xx, wq_ref[...], preferred_element_type=jnp.float32) + br_ref[...]
    o_ref[...] = jnp.maximum(out, 0.0) + jnp.maximum(res, 0.0)


def _k5(x, a0, a1, W_root, W_res, b_conv, b_res):
    BN = 1000
    return pl.pallas_call(
        _k5_body,
        grid=(N // BN,),
        in_specs=[
            pl.BlockSpec((BN, D), lambda i: (i, 0)),
            pl.BlockSpec((BN, H), lambda i: (i, 0)),
            pl.BlockSpec((BN, H), lambda i: (i, 0)),
            pl.BlockSpec((D, H), lambda i: (0, 0)),
            pl.BlockSpec((D, H), lambda i: (0, 0)),
            pl.BlockSpec((1, H), lambda i: (0, 0)),
            pl.BlockSpec((1, H), lambda i: (0, 0)),
        ],
        out_specs=pl.BlockSpec((BN, H), lambda i: (i, 0)),
        out_shape=jax.ShapeDtypeStruct((N, H), jnp.float32),
    )(x, a0, a1, W_root, W_res, b_conv.reshape(1, H), b_res.reshape(1, H))


def kernel(x, edge_index, edge_type, W_rel, W_root, b_conv, W_res, b_res):
    src = edge_index[0]
    dst = edge_index[1]

    hist_all, cntA, cntB = _k1(edge_type, dst)
    ntbl = _kn(cntA, cntB)
    pos, normE, G, blockrel = _k2(hist_all, src, dst, edge_type, ntbl, x)
    M = _k3(blockrel, G, W_rel)
    outp = _k4(pos, dst, M)
    a0 = jnp.concatenate([outp[0:NH], outp[NHALF:NHALF + NH]])
    a1 = jnp.concatenate([outp[2 * NHALF:2 * NHALF + NH],
                          outp[3 * NHALF:3 * NHALF + NH]])
    return _k5(x, a0, a1, W_root, W_res, b_conv, b_res)
